# Initial kernel scaffold; baseline (speedup 1.0000x reference)
import jax
import jax.numpy as jnp
from jax.experimental import pallas as pl


def kernel(node_feats, nodes_of_lanes, l2l_edges, l2l_fused_rpes, params):
    raise NotImplementedError("write your pallas kernel here")


# speedup vs baseline: 1.5933x; 1.5933x over previous
import functools

import jax
import jax.numpy as jnp
import numpy as np
from jax import lax
from jax.experimental import pallas as pl
from jax.experimental.pallas import tpu as pltpu
from jax.experimental.pallas import tpu_sc as plsc

D = 128
H = 8
DH = D // H
L = 10000
NPTS = 100000
E = 320000
IN_DIM = 10
D_RPE = 8
NUM_LAYERS = 2
D_FFN = 2 * D

NC = 2
NS = 16
NW = NC * NS
PLANES = (L + NW - 1) // NW
NEG = np.float32(-3.0e38)

BLK_P = 400
BLK_E = 512
BLK_L = 2000


def _ln(x):
    m = jnp.mean(x, axis=-1, keepdims=True)
    v = jnp.mean((x - m) ** 2, axis=-1, keepdims=True)
    return (x - m) / jnp.sqrt(v + 1e-5)


def _rln(x):
    return jax.nn.relu(_ln(x))


def _cspec(shape):
    return pl.BlockSpec(shape, lambda i, _s=shape: tuple(0 for _ in _s))


def _bspec(blk, width):
    return pl.BlockSpec((blk, width), lambda i: (i, 0))


def _row_call(body, n_rows, blk, in_widths, out_widths, consts):
    grid = (n_rows // blk,)
    in_specs = [_bspec(blk, w) for w in in_widths]
    in_specs += [_cspec(c) for c in consts]
    out_specs = [_bspec(blk, w) for w in out_widths]
    out_shape = [jax.ShapeDtypeStruct((n_rows, w), jnp.float32)
                 for w in out_widths]
    if len(out_widths) == 1:
        out_specs, out_shape = out_specs[0], out_shape[0]
    return pl.pallas_call(body, grid=grid, in_specs=in_specs,
                          out_specs=out_specs, out_shape=out_shape)


_SC_MESH = plsc.VectorSubcoreMesh(core_axis_name="c", subcore_axis_name="s")


def _wid():
    return lax.axis_index("s") * NC + lax.axis_index("c")


def sc_gather(table, idx, n_out, width, chunk=400):
    nch = n_out // (NW * chunk)

    @functools.partial(
        pl.kernel,
        out_type=jax.ShapeDtypeStruct((n_out, width), jnp.float32),
        mesh=_SC_MESH,
        scratch_types=[
            pltpu.VMEM((chunk,), jnp.int32),
            pltpu.VMEM((chunk, width), jnp.float32),
            pltpu.SemaphoreType.DMA,
        ],
    )
    def k(table_hbm, idx_hbm, out_hbm, idx_v, rows_v, sem):
        w = _wid()
        for c in range(nch):
            base = (w * nch + c) * chunk
            pltpu.sync_copy(idx_hbm.at[pl.ds(base, chunk)], idx_v)
            pltpu.async_copy(table_hbm.at[idx_v], rows_v, sem).wait()
            pltpu.sync_copy(rows_v, out_hbm.at[pl.ds(base, chunk)])

    return k(table, idx)


def sc_scatter(rows, idx, n_out, width, chunk=400):
    n_in = rows.shape[0]
    nch = n_in // (NW * chunk)

    @functools.partial(
        pl.kernel,
        out_type=jax.ShapeDtypeStruct((n_out, width), jnp.float32),
        mesh=_SC_MESH,
        scratch_types=[
            pltpu.VMEM((chunk,), jnp.int32),
            pltpu.VMEM((chunk, width), jnp.float32),
            pltpu.SemaphoreType.DMA,
        ],
    )
    def k(rows_hbm, idx_hbm, out_hbm, idx_v, rows_v, sem):
        w = _wid()
        for c in range(nch):
            base = (w * nch + c) * chunk
            pltpu.sync_copy(idx_hbm.at[pl.ds(base, chunk)], idx_v)
            pltpu.sync_copy(rows_hbm.at[pl.ds(base, chunk)], rows_v)
            pltpu.async_copy(rows_v, out_hbm.at[idx_v], sem).wait()

    return k(rows, idx)


def sc_seg_reduce(x, seg_pad, starts, n_rows, width, is_max, expand,
                  chunk=128):
    nfb = width // 16
    out_types = [jax.ShapeDtypeStruct((NW * PLANES * width,), jnp.float32)]
    if expand:
        out_types.append(jax.ShapeDtypeStruct((n_rows + 8, width),
                                              jnp.float32))
    ident = NEG if is_max else np.float32(0.0)
    tab_words = (PLANES + 1) * width

    @functools.partial(
        pl.kernel,
        out_type=tuple(out_types) if expand else out_types[0],
        mesh=_SC_MESH,
        scratch_types=[
            pltpu.VMEM((tab_words,), jnp.float32),
            pltpu.VMEM((chunk * width,), jnp.float32),
            pltpu.VMEM((chunk, width), jnp.float32),
            pltpu.VMEM((chunk + 32,), jnp.int32),
            pltpu.VMEM((chunk,), jnp.int32),
            pltpu.VMEM((NW + 16,), jnp.int32),
            pltpu.SemaphoreType.DMA,
        ],
    )
    def k(x_hbm, seg_hbm, starts_hbm, *rest):
        if expand:
            mx_hbm, y_hbm, tab, buf, obuf, segv, yidx, startv, sem = rest
        else:
            mx_hbm, tab, buf, obuf, segv, yidx, startv, sem = rest
        w = _wid()
        pltpu.sync_copy(starts_hbm, startv)
        my_start = startv[pl.ds(w, 16)][0]
        my_end = startv[pl.ds(w + 1, 16)][0]
        lane0 = w * PLANES
        nchunks = lax.div(my_end - my_start + chunk - 1, chunk)

        def init_body(j, _):
            tab[pl.ds(j * 16, 16)] = jnp.full((16,), ident, jnp.float32)
            return 0
        lax.fori_loop(0, tab_words // 16, init_body, 0)

        def load_seg(cc):
            base_u = my_start + cc * chunk
            base = jnp.maximum(jnp.minimum(base_u, my_end - chunk), 0)
            abase = (base // 8) * 8
            pltpu.sync_copy(seg_hbm.at[pl.ds(abase, chunk + 16)],
                            segv.at[pl.ds(0, chunk + 16)])
            return base_u, base, abase

        def reduce_chunk(cc, _):
            base_u, base, abase = load_seg(cc)
            pltpu.sync_copy(x_hbm.at[pl.ds(base * width, chunk * width)], buf)

            def row(i, _c):
                g = base + i
                sl = segv[pl.ds(i + (base - abase), 16)][0] - lane0
                valid = jnp.logical_and(g >= base_u, g < my_end)
                sl = jnp.where(valid, jnp.clip(sl, 0, PLANES - 1), PLANES)
                for f in range(nfb):
                    t = tab[pl.ds(sl * width + f * 16, 16)]
                    v = buf[pl.ds(i * width + f * 16, 16)]
                    v = jnp.where(valid, v, ident)
                    tab[pl.ds(sl * width + f * 16, 16)] = (
                        jnp.maximum(t, v) if is_max else t + v)
                return _c
            lax.fori_loop(0, chunk, row, 0)
            return 0
        lax.fori_loop(0, nchunks, reduce_chunk, 0)

        if is_max:
            def fix_body(j, _):
                t = tab[pl.ds(j * 16, 16)]
                tab[pl.ds(j * 16, 16)] = jnp.where(t <= NEG, 0.0, t)
                return 0
            lax.fori_loop(0, tab_words // 16, fix_body, 0)

        pltpu.sync_copy(tab.at[pl.ds(0, PLANES * width)],
                        mx_hbm.at[pl.ds(lane0 * width, PLANES * width)])

        if expand:
            def expand_chunk(cc, _):
                base_u, base, abase = load_seg(cc)
                for b in range(chunk // 16):
                    giota = base + b * 16 + lax.iota(jnp.int32, 16)
                    okv = jnp.logical_and(giota >= base_u, giota < my_end)
                    yidx[pl.ds(b * 16, 16)] = jnp.where(okv, giota, n_rows)

                def row(i, _c):
                    sl = jnp.clip(segv[pl.ds(i + (base - abase), 16)][0]
                                  - lane0, 0, PLANES)
                    for f in range(nfb):
                        obuf[i, pl.ds(f * 16, 16)] = tab[
                            pl.ds(sl * width + f * 16, 16)]
                    return _c
                lax.fori_loop(0, chunk, row, 0)
                pltpu.async_copy(obuf, y_hbm.at[yidx], sem).wait()
                return 0
            lax.fori_loop(0, nchunks, expand_chunk, 0)

    outs = k(x.reshape(-1), seg_pad, starts)
    if expand:
        mx, y = outs
        return mx.reshape(NW * PLANES, width)[:L], y[:n_rows]
    return outs.reshape(NW * PLANES, width)[:L]


def sc_attn_aggregate(lg, v, seg_pad, starts, chunk=128):
    tabw = (PLANES + 1) * 16
    taga = (PLANES + 1) * D

    @functools.partial(
        pl.kernel,
        out_type=jax.ShapeDtypeStruct((NW * PLANES * D,), jnp.float32),
        mesh=_SC_MESH,
        scratch_types=[
            pltpu.VMEM((tabw,), jnp.float32),
            pltpu.VMEM((tabw,), jnp.float32),
            pltpu.VMEM((taga,), jnp.float32),
            pltpu.VMEM((chunk * 16,), jnp.float32),
            pltpu.VMEM((chunk * D,), jnp.float32),
            pltpu.VMEM((chunk + 32,), jnp.int32),
            pltpu.VMEM((NW + 16,), jnp.int32),
        ],
    )
    def k(lg_hbm, v_hbm, seg_hbm, starts_hbm, out_hbm,
          mtab, stab, atab, lbuf, vbuf, segv, startv):
        w = _wid()
        pltpu.sync_copy(starts_hbm, startv)
        my_start = startv[pl.ds(w, 16)][0]
        my_end = startv[pl.ds(w + 1, 16)][0]
        lane0 = w * PLANES
        nchunks = lax.div(my_end - my_start + chunk - 1, chunk)

        def init16(j, _):
            mtab[pl.ds(j * 16, 16)] = jnp.full((16,), NEG, jnp.float32)
            stab[pl.ds(j * 16, 16)] = jnp.zeros((16,), jnp.float32)
            return 0
        lax.fori_loop(0, tabw // 16, init16, 0)

        def inita(j, _):
            atab[pl.ds(j * 16, 16)] = jnp.zeros((16,), jnp.float32)
            return 0
        lax.fori_loop(0, taga // 16, inita, 0)

        def load_seg(cc):
            base_u = my_start + cc * chunk
            base = jnp.maximum(jnp.minimum(base_u, my_end - chunk), 0)
            abase = (base // 8) * 8
            pltpu.sync_copy(seg_hbm.at[pl.ds(abase, chunk + 16)],
                            segv.at[pl.ds(0, chunk + 16)])
            return base_u, base, abase

        def pass_mx(cc, _):
            base_u, base, abase = load_seg(cc)
            pltpu.sync_copy(lg_hbm.at[pl.ds(base * 16, chunk * 16)], lbuf)

            def row(i, _c):
                g = base + i
                sl = segv[pl.ds(i + (base - abase), 16)][0] - lane0
                valid = jnp.logical_and(g >= base_u, g < my_end)
                sl = jnp.where(valid, jnp.clip(sl, 0, PLANES - 1), PLANES)
                t = mtab[pl.ds(sl * 16, 16)]
                lv = jnp.where(valid, lbuf[pl.ds(i * 16, 16)], NEG)
                mtab[pl.ds(sl * 16, 16)] = jnp.maximum(t, lv)
                return _c
            lax.fori_loop(0, chunk, row, 0)
            return 0
        lax.fori_loop(0, nchunks, pass_mx, 0)

        def pass_sum(cc, _):
            base_u, base, abase = load_seg(cc)
            pltpu.sync_copy(lg_hbm.at[pl.ds(base * 16, chunk * 16)], lbuf)

            def row(i, _c):
                g = base + i
                sl = segv[pl.ds(i + (base - abase), 16)][0] - lane0
                valid = jnp.logical_and(g >= base_u, g < my_end)
                sl = jnp.where(valid, jnp.clip(sl, 0, PLANES - 1), PLANES)
                ev = jnp.exp(lbuf[pl.ds(i * 16, 16)] - mtab[pl.ds(sl * 16, 16)])
                ev = jnp.where(valid, ev, 0.0)
                stab[pl.ds(sl * 16, 16)] = stab[pl.ds(sl * 16, 16)] + ev
                return _c
            lax.fori_loop(0, chunk, row, 0)
            return 0
        lax.fori_loop(0, nchunks, pass_sum, 0)

        def pass_wv(cc, _):
            base_u, base, abase = load_seg(cc)
            pltpu.sync_copy(lg_hbm.at[pl.ds(base * 16, chunk * 16)], lbuf)
            pltpu.sync_copy(v_hbm.at[pl.ds(base * D, chunk * D)], vbuf)

            def row(i, _c):
                g = base + i
                sl = segv[pl.ds(i + (base - abase), 16)][0] - lane0
                valid = jnp.logical_and(g >= base_u, g < my_end)
                sl = jnp.where(valid, jnp.clip(sl, 0, PLANES - 1), PLANES)
                ev = jnp.exp(lbuf[pl.ds(i * 16, 16)] - mtab[pl.ds(sl * 16, 16)])
                wt = ev / (stab[pl.ds(sl * 16, 16)] + 1e-16)
                wt = jnp.where(valid, wt, 0.0)
                for h in range(H):
                    a = atab[pl.ds(sl * D + h * 16, 16)]
                    vv = vbuf[pl.ds(i * D + h * 16, 16)]
                    atab[pl.ds(sl * D + h * 16, 16)] = a + vv * wt[h]
                return _c
            lax.fori_loop(0, chunk, row, 0)
            return 0
        lax.fori_loop(0, nchunks, pass_wv, 0)

        pltpu.sync_copy(atab.at[pl.ds(0, PLANES * D)],
                        out_hbm.at[pl.ds(lane0 * D, PLANES * D)])

    out = k(lg.reshape(-1), v.reshape(-1), seg_pad, starts)
    return out.reshape(NW * PLANES, D)[:L]


def _points1_body(nf, projW, projB, f1aW, f1aB, f1bW, f1bB, x0o, x1o):
    x0 = _rln(jnp.dot(nf[...], projW[...]) + projB[...])
    t = _rln(jnp.dot(x0, f1aW[...]) + f1aB[...])
    x1 = _rln(jnp.dot(t, f1bW[...]) + f1bB[...])
    x0o[...] = x0
    x1o[...] = x1


def _fc2_body(xin, x1, y1, aWt, aWb, aB, bW, bB, f1aW, f1aB, f1bW, f1bB,
              outo, x2o):
    h = _rln(jnp.dot(x1[...], aWt[...]) + jnp.dot(y1[...], aWb[...]) + aB[...])
    h = _rln(jnp.dot(h, bW[...]) + bB[...])
    out = _ln(xin[...] + h)
    t = _rln(jnp.dot(out, f1aW[...]) + f1aB[...])
    x2 = _rln(jnp.dot(t, f1bW[...]) + f1bB[...])
    outo[...] = out
    x2o[...] = x2


def _fc2_final_body(xin, x1, y1, aWt, aWb, aB, bW, bB, outo):
    h = _rln(jnp.dot(x1[...], aWt[...]) + jnp.dot(y1[...], aWb[...]) + aB[...])
    h = _rln(jnp.dot(h, bW[...]) + bB[...])
    outo[...] = _ln(xin[...] + h)


def _rpe_body(rp, rW, rB, eao):
    eao[...] = _rln(jnp.dot(rp[...], rW[...]) + rB[...])


def _edge1_body(xd, xs, ea, W1, W2, W3, mB, euW, euB, qW, kW, vW, hsel,
                eao, vo, lo):
    mem = _rln(jnp.dot(xd[...], W1[...]) + jnp.dot(xs[...], W2[...])
               + jnp.dot(ea[...], W3[...]) + mB[...])
    delta = _rln(jnp.dot(mem, euW[...]) + euB[...])
    eao[...] = _ln(ea[...] + delta)
    q = jnp.dot(xd[...], qW[...])
    kk = jnp.dot(mem, kW[...])
    vo[...] = jnp.dot(mem, vW[...])
    lo[...] = jnp.dot(q * kk, hsel[...]) * (1.0 / np.sqrt(DH))


def _lane_body(x, aggr, oW, f1W, f1B, f2W, f2B, xo):
    x1 = _ln(x[...] + jnp.dot(aggr[...], oW[...]))
    h = jax.nn.relu(jnp.dot(x1, f1W[...]) + f1B[...])
    xo[...] = _ln(x1 + jnp.dot(h, f2W[...]) + f2B[...])


def kernel(node_feats, nodes_of_lanes, l2l_edges, l2l_fused_rpes, params):
    p = params
    f32 = jnp.float32

    src, dst = l2l_edges[0], l2l_edges[1]
    perm = jnp.argsort(dst).astype(jnp.int32)
    dst_s = dst[perm]
    src_s = src[perm]
    lane_cuts = jnp.arange(NW + 16, dtype=jnp.int32) * PLANES
    e_starts = jnp.minimum(
        jnp.searchsorted(dst_s, lane_cuts), E).astype(jnp.int32)
    n_starts = jnp.minimum(
        jnp.searchsorted(nodes_of_lanes, lane_cuts), NPTS).astype(jnp.int32)
    nol_pad = jnp.concatenate([nodes_of_lanes, jnp.full((16,), L, jnp.int32)])
    dst_pad = jnp.concatenate([dst_s, jnp.full((16,), L, jnp.int32)])

    nf_pad = jnp.zeros((NPTS, D), f32).at[:, :IN_DIM].set(node_feats)
    projW = jnp.zeros((D, D), f32).at[:IN_DIM].set(p['proj_W'])

    shp_dd = (D, D)
    x0, x1 = _row_call(
        _points1_body, NPTS, BLK_P, [D], [D, D],
        consts=(shp_dd, (D,), shp_dd, (D,), shp_dd, (D,)),
    )(nf_pad, projW, p['proj_b'], p['pa_fc1a_W'], p['pa_fc1a_b'],
      p['pa_fc1b_W'], p['pa_fc1b_b'])

    _, y1 = sc_seg_reduce(x1, nol_pad, n_starts, NPTS, D,
                          is_max=True, expand=True)

    out_pa, x2 = _row_call(
        _fc2_body, NPTS, BLK_P, [D, D, D], [D, D],
        consts=(shp_dd, shp_dd, (D,), shp_dd, (D,), shp_dd, (D,),
                shp_dd, (D,)),
    )(x0, x1, y1, p['pa_fc2a_W'][:D], p['pa_fc2a_W'][D:], p['pa_fc2a_b'],
      p['pa_fc2b_W'], p['pa_fc2b_b'],
      p['la_fc1a_W'], p['la_fc1a_b'], p['la_fc1b_W'], p['la_fc1b_b'])

    _, y2 = sc_seg_reduce(x2, nol_pad, n_starts, NPTS, D,
                          is_max=True, expand=True)

    out_la = _row_call(
        _fc2_final_body, NPTS, BLK_P, [D, D, D], [D],
        consts=(shp_dd, shp_dd, (D,), shp_dd, (D,)),
    )(out_pa, x2, y2, p['la_fc2a_W'][:D], p['la_fc2a_W'][D:],
      p['la_fc2a_b'], p['la_fc2b_W'], p['la_fc2b_b'])

    x = sc_seg_reduce(out_la, nol_pad, n_starts, NPTS, D,
                      is_max=True, expand=False)

    rpW = jnp.zeros((16, D), f32).at[:D_RPE].set(p['rpe_W'])
    rpes_p = jnp.zeros((E, 16), f32).at[:, :D_RPE].set(l2l_fused_rpes)
    ea0 = _row_call(
        _rpe_body, E, BLK_E, [16], [D],
        consts=((16, D), (D,)),
    )(rpes_p, rpW, p['rpe_b'])
    ea = sc_gather(ea0, perm, E, D)

    hsel_np = np.zeros((D, 16), np.float32)
    for h in range(H):
        hsel_np[h * DH:(h + 1) * DH, h] = 1.0
    hsel = jnp.asarray(hsel_np)

    for l in range(NUM_LAYERS):
        pre = 'l%d_' % l
        mW = p[pre + 'mem_W']
        xd = sc_gather(x, dst_s, E, D)
        xs = sc_gather(x, src_s, E, D)
        ea, v, lg = pl.pallas_call(
            _edge1_body,
            grid=(E // BLK_E,),
            in_specs=[_bspec(BLK_E, D)] * 3 + [
                _cspec(shp_dd), _cspec(shp_dd), _cspec(shp_dd), _cspec((D,)),
                _cspec(shp_dd), _cspec((D,)), _cspec(shp_dd), _cspec(shp_dd),
                _cspec(shp_dd), _cspec((D, 16)),
            ],
            out_specs=[_bspec(BLK_E, D), _bspec(BLK_E, D), _bspec(BLK_E, 16)],
            out_shape=[jax.ShapeDtypeStruct((E, D), f32),
                       jax.ShapeDtypeStruct((E, D), f32),
                       jax.ShapeDtypeStruct((E, 16), f32)],
        )(xd, xs, ea, mW[:D], mW[D:2 * D], mW[2 * D:], p[pre + 'mem_b'],
          p[pre + 'eu_W'], p[pre + 'eu_b'], p[pre + 'q_W'], p[pre + 'k_W'],
          p[pre + 'v_W'], hsel)

        aggr = sc_attn_aggregate(lg, v, dst_pad, e_starts)

        x = _row_call(
            _lane_body, L, BLK_L, [D, D], [D],
            consts=(shp_dd, (D, D_FFN), (D_FFN,), (D_FFN, D), (D,)),
        )(x, aggr, p[pre + 'o_W'], p[pre + 'ffn1_W'], p[pre + 'ffn1_b'],
          p[pre + 'ffn2_W'], p[pre + 'ffn2_b'])

    l2l_attr = sc_scatter(ea, perm, E, D)
    return (x, l2l_attr)


# speedup vs baseline: 1.8458x; 1.1584x over previous
import functools

import jax
import jax.numpy as jnp
import numpy as np
from jax import lax
from jax.experimental import pallas as pl
from jax.experimental.pallas import tpu as pltpu
from jax.experimental.pallas import tpu_sc as plsc

D = 128
H = 8
DH = D // H
L = 10000
NPTS = 100000
E = 320000
IN_DIM = 10
D_RPE = 8
NUM_LAYERS = 2
D_FFN = 2 * D

NC = 2
NS = 16
NW = NC * NS
PLANES = (L + NW - 1) // NW
NEG = np.float32(-3.0e38)

BLK_P = 400
BLK_E = 512
BLK_L = 2000


def _ln(x):
    m = jnp.mean(x, axis=-1, keepdims=True)
    v = jnp.mean((x - m) ** 2, axis=-1, keepdims=True)
    return (x - m) / jnp.sqrt(v + 1e-5)


def _rln(x):
    return jax.nn.relu(_ln(x))


def _cspec(shape):
    return pl.BlockSpec(shape, lambda i, _s=shape: tuple(0 for _ in _s))


def _bspec(blk, width):
    return pl.BlockSpec((blk, width), lambda i: (i, 0))


def _row_call(body, n_rows, blk, in_widths, out_widths, consts):
    grid = (n_rows // blk,)
    in_specs = [_bspec(blk, w) for w in in_widths]
    in_specs += [_cspec(c) for c in consts]
    out_specs = [_bspec(blk, w) for w in out_widths]
    out_shape = [jax.ShapeDtypeStruct((n_rows, w), jnp.float32)
                 for w in out_widths]
    if len(out_widths) == 1:
        out_specs, out_shape = out_specs[0], out_shape[0]
    return pl.pallas_call(body, grid=grid, in_specs=in_specs,
                          out_specs=out_specs, out_shape=out_shape)


_SC_MESH = plsc.VectorSubcoreMesh(core_axis_name="c", subcore_axis_name="s")


def _wid():
    return lax.axis_index("s") * NC + lax.axis_index("c")


def sc_gather(table, idx, n_out, width, chunk=400):
    nch = n_out // (NW * chunk)

    @functools.partial(
        pl.kernel,
        out_type=jax.ShapeDtypeStruct((n_out, width), jnp.float32),
        mesh=_SC_MESH,
        scratch_types=[
            pltpu.VMEM((chunk,), jnp.int32),
            pltpu.VMEM((chunk, width), jnp.float32),
            pltpu.SemaphoreType.DMA,
        ],
    )
    def k(table_hbm, idx_hbm, out_hbm, idx_v, rows_v, sem):
        w = _wid()
        for c in range(nch):
            base = (w * nch + c) * chunk
            pltpu.sync_copy(idx_hbm.at[pl.ds(base, chunk)], idx_v)
            pltpu.async_copy(table_hbm.at[idx_v], rows_v, sem).wait()
            pltpu.sync_copy(rows_v, out_hbm.at[pl.ds(base, chunk)])

    return k(table, idx)


def sc_scatter(rows, idx, n_out, width, chunk=400):
    n_in = rows.shape[0]
    nch = n_in // (NW * chunk)

    @functools.partial(
        pl.kernel,
        out_type=jax.ShapeDtypeStruct((n_out, width), jnp.float32),
        mesh=_SC_MESH,
        scratch_types=[
            pltpu.VMEM((chunk,), jnp.int32),
            pltpu.VMEM((chunk, width), jnp.float32),
            pltpu.SemaphoreType.DMA,
        ],
    )
    def k(rows_hbm, idx_hbm, out_hbm, idx_v, rows_v, sem):
        w = _wid()
        for c in range(nch):
            base = (w * nch + c) * chunk
            pltpu.sync_copy(idx_hbm.at[pl.ds(base, chunk)], idx_v)
            pltpu.sync_copy(rows_hbm.at[pl.ds(base, chunk)], rows_v)
            pltpu.async_copy(rows_v, out_hbm.at[idx_v], sem).wait()

    return k(rows, idx)


def sc_seg_reduce(x, seg_pad, starts, n_rows, width, is_max, expand,
                  chunk=128):
    nfb = width // 16
    out_types = [jax.ShapeDtypeStruct((NW * PLANES * width,), jnp.float32)]
    if expand:
        out_types.append(jax.ShapeDtypeStruct((n_rows + 8, width),
                                              jnp.float32))
    ident = NEG if is_max else np.float32(0.0)
    tab_words = (PLANES + 2) * width

    @functools.partial(
        pl.kernel,
        out_type=tuple(out_types) if expand else out_types[0],
        mesh=_SC_MESH,
        scratch_types=[
            pltpu.VMEM((tab_words,), jnp.float32),
            pltpu.VMEM((chunk * width,), jnp.float32),
            pltpu.VMEM((chunk, width), jnp.float32),
            pltpu.VMEM((chunk + 32,), jnp.int32),
            pltpu.VMEM((chunk,), jnp.int32),
            pltpu.VMEM((NW + 16,), jnp.int32),
            pltpu.SemaphoreType.DMA,
        ],
    )
    def k(x_hbm, seg_hbm, starts_hbm, *rest):
        if expand:
            mx_hbm, y_hbm, tab, buf, obuf, segv, yidx, startv, sem = rest
        else:
            mx_hbm, tab, buf, obuf, segv, yidx, startv, sem = rest
        w = _wid()
        pltpu.sync_copy(starts_hbm, startv)
        my_start = startv[pl.ds(w, 16)][0]
        my_end = startv[pl.ds(w + 1, 16)][0]
        lane0 = w * PLANES
        nchunks = lax.div(my_end - my_start + chunk - 1, chunk)

        def init_body(j, _):
            tab[pl.ds(j * 16, 16)] = jnp.full((16,), ident, jnp.float32)
            return 0
        lax.fori_loop(0, tab_words // 16, init_body, 0)

        def load_seg(cc):
            base_u = my_start + cc * chunk
            base = jnp.maximum(jnp.minimum(base_u, my_end - chunk), 0)
            abase = (base // 8) * 8
            pltpu.sync_copy(seg_hbm.at[pl.ds(abase, chunk + 16)],
                            segv.at[pl.ds(0, chunk + 16)])
            return base_u, base, abase

        def _comb(a, b):
            return jnp.maximum(a, b) if is_max else a + b

        def reduce_chunk(cc, _):
            base_u, base, abase = load_seg(cc)
            pltpu.sync_copy(x_hbm.at[pl.ds(base * width, chunk * width)], buf)

            def row(i, carry):
                cur = carry[0]
                acc = carry[1:]
                g = base + i
                sl = segv[pl.ds(i + (base - abase), 16)][0] - lane0
                valid = jnp.logical_and(g >= base_u, g < my_end)
                sl = jnp.where(valid, sl, PLANES)
                changed = sl != cur

                def flush(_):
                    for f in range(nfb):
                        t = tab[pl.ds(cur * width + f * 16, 16)]
                        tab[pl.ds(cur * width + f * 16, 16)] = _comb(t, acc[f])
                    return 0
                lax.cond(changed, flush, lambda _: 0, 0)
                acc3 = tuple(
                    _comb(jnp.where(changed, ident, acc[f]),
                          buf[pl.ds(i * width + f * 16, 16)])
                    for f in range(nfb))
                return (sl,) + acc3

            iacc = tuple(jnp.full((16,), ident, jnp.float32)
                         for _ in range(nfb))
            fin = lax.fori_loop(0, chunk, row, (PLANES + 1,) + iacc)
            cur = fin[0]
            for f in range(nfb):
                t = tab[pl.ds(cur * width + f * 16, 16)]
                tab[pl.ds(cur * width + f * 16, 16)] = _comb(t, fin[1 + f])
            return 0
        lax.fori_loop(0, nchunks, reduce_chunk, 0)

        if is_max:
            def fix_body(j, _):
                t = tab[pl.ds(j * 16, 16)]
                tab[pl.ds(j * 16, 16)] = jnp.where(t <= NEG, 0.0, t)
                return 0
            lax.fori_loop(0, tab_words // 16, fix_body, 0)

        pltpu.sync_copy(tab.at[pl.ds(0, PLANES * width)],
                        mx_hbm.at[pl.ds(lane0 * width, PLANES * width)])

        if expand:
            def expand_chunk(cc, _):
                base_u, base, abase = load_seg(cc)
                for b in range(chunk // 16):
                    giota = base + b * 16 + lax.iota(jnp.int32, 16)
                    okv = jnp.logical_and(giota >= base_u, giota < my_end)
                    yidx[pl.ds(b * 16, 16)] = jnp.where(okv, giota, n_rows)

                def row(i, _c):
                    sl = jnp.clip(segv[pl.ds(i + (base - abase), 16)][0]
                                  - lane0, 0, PLANES)
                    for f in range(nfb):
                        obuf[i, pl.ds(f * 16, 16)] = tab[
                            pl.ds(sl * width + f * 16, 16)]
                    return _c

                lax.fori_loop(0, chunk, row, 0)
                pltpu.async_copy(obuf, y_hbm.at[yidx], sem).wait()
                return 0
            lax.fori_loop(0, nchunks, expand_chunk, 0)

    outs = k(x.reshape(-1), seg_pad, starts)
    if expand:
        mx, y = outs
        return mx.reshape(NW * PLANES, width)[:L], y[:n_rows]
    return outs.reshape(NW * PLANES, width)[:L]


def sc_attn_aggregate(lg, v, seg_pad, starts, chunk=128):
    tabw = (PLANES + 2) * 16
    taga = (PLANES + 2) * D

    @functools.partial(
        pl.kernel,
        out_type=jax.ShapeDtypeStruct((NW * PLANES * D,), jnp.float32),
        mesh=_SC_MESH,
        scratch_types=[
            pltpu.VMEM((tabw,), jnp.float32),
            pltpu.VMEM((tabw,), jnp.float32),
            pltpu.VMEM((taga,), jnp.float32),
            pltpu.VMEM((chunk * 16,), jnp.float32),
            pltpu.VMEM((chunk * D,), jnp.float32),
            pltpu.VMEM((chunk + 32,), jnp.int32),
            pltpu.VMEM((NW + 16,), jnp.int32),
        ],
    )
    def k(lg_hbm, v_hbm, seg_hbm, starts_hbm, out_hbm,
          mtab, stab, atab, lbuf, vbuf, segv, startv):
        w = _wid()
        pltpu.sync_copy(starts_hbm, startv)
        my_start = startv[pl.ds(w, 16)][0]
        my_end = startv[pl.ds(w + 1, 16)][0]
        lane0 = w * PLANES
        nchunks = lax.div(my_end - my_start + chunk - 1, chunk)

        def init16(j, _):
            mtab[pl.ds(j * 16, 16)] = jnp.full((16,), NEG, jnp.float32)
            stab[pl.ds(j * 16, 16)] = jnp.zeros((16,), jnp.float32)
            return 0
        lax.fori_loop(0, tabw // 16, init16, 0)

        def inita(j, _):
            atab[pl.ds(j * 16, 16)] = jnp.zeros((16,), jnp.float32)
            return 0
        lax.fori_loop(0, taga // 16, inita, 0)

        def load_seg(cc):
            base_u = my_start + cc * chunk
            base = jnp.maximum(jnp.minimum(base_u, my_end - chunk), 0)
            abase = (base // 8) * 8
            pltpu.sync_copy(seg_hbm.at[pl.ds(abase, chunk + 16)],
                            segv.at[pl.ds(0, chunk + 16)])
            return base_u, base, abase

        def rowmeta(base_u, base, abase, i):
            g = base + i
            sl = segv[pl.ds(i + (base - abase), 16)][0] - lane0
            valid = jnp.logical_and(g >= base_u, g < my_end)
            return jnp.where(valid, sl, PLANES)

        def pass_mx(cc, _):
            base_u, base, abase = load_seg(cc)
            pltpu.sync_copy(lg_hbm.at[pl.ds(base * 16, chunk * 16)], lbuf)

            def row(i, carry):
                cur, acc = carry
                sl = rowmeta(base_u, base, abase, i)
                changed = sl != cur

                def flush(_):
                    t = mtab[pl.ds(cur * 16, 16)]
                    mtab[pl.ds(cur * 16, 16)] = jnp.maximum(t, acc)
                    return 0
                lax.cond(changed, flush, lambda _: 0, 0)
                acc2 = jnp.where(changed, NEG, acc)
                return (sl, jnp.maximum(acc2, lbuf[pl.ds(i * 16, 16)]))

            cur, acc = lax.fori_loop(
                0, chunk, row,
                (PLANES + 1, jnp.full((16,), NEG, jnp.float32)))
            t = mtab[pl.ds(cur * 16, 16)]
            mtab[pl.ds(cur * 16, 16)] = jnp.maximum(t, acc)
            return 0
        lax.fori_loop(0, nchunks, pass_mx, 0)

        def pass_sum(cc, _):
            base_u, base, abase = load_seg(cc)
            pltpu.sync_copy(lg_hbm.at[pl.ds(base * 16, chunk * 16)], lbuf)

            def row(i, carry):
                cur, acc = carry
                sl = rowmeta(base_u, base, abase, i)
                changed = sl != cur

                def flush(_):
                    t = stab[pl.ds(cur * 16, 16)]
                    stab[pl.ds(cur * 16, 16)] = t + acc
                    return 0
                lax.cond(changed, flush, lambda _: 0, 0)
                acc2 = jnp.where(changed, 0.0, acc)
                mv = mtab[pl.ds(sl * 16, 16)]
                ev = jnp.exp(lbuf[pl.ds(i * 16, 16)] - mv)
                return (sl, acc2 + ev)

            cur, acc = lax.fori_loop(
                0, chunk, row,
                (PLANES + 1, jnp.zeros((16,), jnp.float32)))
            t = stab[pl.ds(cur * 16, 16)]
            stab[pl.ds(cur * 16, 16)] = t + acc
            return 0
        lax.fori_loop(0, nchunks, pass_sum, 0)

        def pass_wv(cc, _):
            base_u, base, abase = load_seg(cc)
            pltpu.sync_copy(lg_hbm.at[pl.ds(base * 16, chunk * 16)], lbuf)
            pltpu.sync_copy(v_hbm.at[pl.ds(base * D, chunk * D)], vbuf)

            def row(i, carry):
                cur = carry[0]
                acc = carry[1:]
                sl = rowmeta(base_u, base, abase, i)
                changed = sl != cur

                def flush(_):
                    for h in range(H):
                        a = atab[pl.ds(cur * D + h * 16, 16)]
                        atab[pl.ds(cur * D + h * 16, 16)] = a + acc[h]
                    return 0
                lax.cond(changed, flush, lambda _: 0, 0)
                mv = mtab[pl.ds(sl * 16, 16)]
                sv = stab[pl.ds(sl * 16, 16)]
                wt = jnp.exp(lbuf[pl.ds(i * 16, 16)] - mv) / (sv + 1e-16)
                acc3 = tuple(
                    jnp.where(changed, 0.0, acc[h])
                    + vbuf[pl.ds(i * D + h * 16, 16)] * wt[h]
                    for h in range(H))
                return (sl,) + acc3

            zero = jnp.zeros((16,), jnp.float32)
            fin = lax.fori_loop(
                0, chunk, row,
                (PLANES + 1,) + tuple(zero for _ in range(H)))
            cur = fin[0]
            for h in range(H):
                a = atab[pl.ds(cur * D + h * 16, 16)]
                atab[pl.ds(cur * D + h * 16, 16)] = a + fin[1 + h]
            return 0
        lax.fori_loop(0, nchunks, pass_wv, 0)

        pltpu.sync_copy(atab.at[pl.ds(0, PLANES * D)],
                        out_hbm.at[pl.ds(lane0 * D, PLANES * D)])

    out = k(lg.reshape(-1), v.reshape(-1), seg_pad, starts)
    return out.reshape(NW * PLANES, D)[:L]


def _points1_body(nf, projW, projB, f1aW, f1aB, f1bW, f1bB, x0o, x1o):
    x0 = _rln(jnp.dot(nf[...], projW[...]) + projB[...])
    t = _rln(jnp.dot(x0, f1aW[...]) + f1aB[...])
    x1 = _rln(jnp.dot(t, f1bW[...]) + f1bB[...])
    x0o[...] = x0
    x1o[...] = x1


def _fc2_body(xin, x1, y1, aWt, aWb, aB, bW, bB, f1aW, f1aB, f1bW, f1bB,
              outo, x2o):
    h = _rln(jnp.dot(x1[...], aWt[...]) + jnp.dot(y1[...], aWb[...]) + aB[...])
    h = _rln(jnp.dot(h, bW[...]) + bB[...])
    out = _ln(xin[...] + h)
    t = _rln(jnp.dot(out, f1aW[...]) + f1aB[...])
    x2 = _rln(jnp.dot(t, f1bW[...]) + f1bB[...])
    outo[...] = out
    x2o[...] = x2


def _fc2_final_body(xin, x1, y1, aWt, aWb, aB, bW, bB, outo):
    h = _rln(jnp.dot(x1[...], aWt[...]) + jnp.dot(y1[...], aWb[...]) + aB[...])
    h = _rln(jnp.dot(h, bW[...]) + bB[...])
    outo[...] = _ln(xin[...] + h)


def _rpe_body(rp, rW, rB, eao):
    eao[...] = _rln(jnp.dot(rp[...], rW[...]) + rB[...])


def _edge1_body(xd, xs, ea, W1, W2, W3, mB, euW, euB, qW, kW, vW, hsel,
                eao, vo, lo):
    mem = _rln(jnp.dot(xd[...], W1[...]) + jnp.dot(xs[...], W2[...])
               + jnp.dot(ea[...], W3[...]) + mB[...])
    delta = _rln(jnp.dot(mem, euW[...]) + euB[...])
    eao[...] = _ln(ea[...] + delta)
    q = jnp.dot(xd[...], qW[...])
    kk = jnp.dot(mem, kW[...])
    vo[...] = jnp.dot(mem, vW[...])
    lo[...] = jnp.dot(q * kk, hsel[...]) * (1.0 / np.sqrt(DH))


def _lane_body(x, aggr, oW, f1W, f1B, f2W, f2B, xo):
    x1 = _ln(x[...] + jnp.dot(aggr[...], oW[...]))
    h = jax.nn.relu(jnp.dot(x1, f1W[...]) + f1B[...])
    xo[...] = _ln(x1 + jnp.dot(h, f2W[...]) + f2B[...])


def kernel(node_feats, nodes_of_lanes, l2l_edges, l2l_fused_rpes, params):
    p = params
    f32 = jnp.float32

    src, dst = l2l_edges[0], l2l_edges[1]
    perm = jnp.argsort(dst).astype(jnp.int32)
    dst_s = dst[perm]
    src_s = src[perm]
    lane_cuts = jnp.arange(NW + 16, dtype=jnp.int32) * PLANES
    e_starts = jnp.minimum(
        jnp.searchsorted(dst_s, lane_cuts), E).astype(jnp.int32)
    n_starts = jnp.minimum(
        jnp.searchsorted(nodes_of_lanes, lane_cuts), NPTS).astype(jnp.int32)
    nol_pad = jnp.concatenate([nodes_of_lanes, jnp.full((16,), L, jnp.int32)])
    dst_pad = jnp.concatenate([dst_s, jnp.full((16,), L, jnp.int32)])

    nf_pad = jnp.zeros((NPTS, D), f32).at[:, :IN_DIM].set(node_feats)
    projW = jnp.zeros((D, D), f32).at[:IN_DIM].set(p['proj_W'])

    shp_dd = (D, D)
    x0, x1 = _row_call(
        _points1_body, NPTS, BLK_P, [D], [D, D],
        consts=(shp_dd, (D,), shp_dd, (D,), shp_dd, (D,)),
    )(nf_pad, projW, p['proj_b'], p['pa_fc1a_W'], p['pa_fc1a_b'],
      p['pa_fc1b_W'], p['pa_fc1b_b'])

    _, y1 = sc_seg_reduce(x1, nol_pad, n_starts, NPTS, D,
                          is_max=True, expand=True)

    out_pa, x2 = _row_call(
        _fc2_body, NPTS, BLK_P, [D, D, D], [D, D],
        consts=(shp_dd, shp_dd, (D,), shp_dd, (D,), shp_dd, (D,),
                shp_dd, (D,)),
    )(x0, x1, y1, p['pa_fc2a_W'][:D], p['pa_fc2a_W'][D:], p['pa_fc2a_b'],
      p['pa_fc2b_W'], p['pa_fc2b_b'],
      p['la_fc1a_W'], p['la_fc1a_b'], p['la_fc1b_W'], p['la_fc1b_b'])

    _, y2 = sc_seg_reduce(x2, nol_pad, n_starts, NPTS, D,
                          is_max=True, expand=True)

    out_la = _row_call(
        _fc2_final_body, NPTS, BLK_P, [D, D, D], [D],
        consts=(shp_dd, shp_dd, (D,), shp_dd, (D,)),
    )(out_pa, x2, y2, p['la_fc2a_W'][:D], p['la_fc2a_W'][D:],
      p['la_fc2a_b'], p['la_fc2b_W'], p['la_fc2b_b'])

    x = sc_seg_reduce(out_la, nol_pad, n_starts, NPTS, D,
                      is_max=True, expand=False)

    rpW = jnp.zeros((16, D), f32).at[:D_RPE].set(p['rpe_W'])
    rpes_p = jnp.zeros((E, 16), f32).at[:, :D_RPE].set(l2l_fused_rpes)
    ea0 = _row_call(
        _rpe_body, E, BLK_E, [16], [D],
        consts=((16, D), (D,)),
    )(rpes_p, rpW, p['rpe_b'])
    ea = sc_gather(ea0, perm, E, D)

    hsel_np = np.zeros((D, 16), np.float32)
    for h in range(H):
        hsel_np[h * DH:(h + 1) * DH, h] = 1.0
    hsel = jnp.asarray(hsel_np)

    for l in range(NUM_LAYERS):
        pre = 'l%d_' % l
        mW = p[pre + 'mem_W']
        xd = sc_gather(x, dst_s, E, D)
        xs = sc_gather(x, src_s, E, D)
        ea, v, lg = pl.pallas_call(
            _edge1_body,
            grid=(E // BLK_E,),
            in_specs=[_bspec(BLK_E, D)] * 3 + [
                _cspec(shp_dd), _cspec(shp_dd), _cspec(shp_dd), _cspec((D,)),
                _cspec(shp_dd), _cspec((D,)), _cspec(shp_dd), _cspec(shp_dd),
                _cspec(shp_dd), _cspec((D, 16)),
            ],
            out_specs=[_bspec(BLK_E, D), _bspec(BLK_E, D), _bspec(BLK_E, 16)],
            out_shape=[jax.ShapeDtypeStruct((E, D), f32),
                       jax.ShapeDtypeStruct((E, D), f32),
                       jax.ShapeDtypeStruct((E, 16), f32)],
        )(xd, xs, ea, mW[:D], mW[D:2 * D], mW[2 * D:], p[pre + 'mem_b'],
          p[pre + 'eu_W'], p[pre + 'eu_b'], p[pre + 'q_W'], p[pre + 'k_W'],
          p[pre + 'v_W'], hsel)

        aggr = sc_attn_aggregate(lg, v, dst_pad, e_starts)

        x = _row_call(
            _lane_body, L, BLK_L, [D, D], [D],
            consts=(shp_dd, (D, D_FFN), (D_FFN,), (D_FFN, D), (D,)),
        )(x, aggr, p[pre + 'o_W'], p[pre + 'ffn1_W'], p[pre + 'ffn1_b'],
          p[pre + 'ffn2_W'], p[pre + 'ffn2_b'])

    l2l_attr = sc_scatter(ea, perm, E, D)
    return (x, l2l_attr)


# speedup vs baseline: 1.9012x; 1.0300x over previous
import functools

import jax
import jax.numpy as jnp
import numpy as np
from jax import lax
from jax.experimental import pallas as pl
from jax.experimental.pallas import tpu as pltpu
from jax.experimental.pallas import tpu_sc as plsc

D = 128
H = 8
DH = D // H
L = 10000
NPTS = 100000
E = 320000
IN_DIM = 10
D_RPE = 8
NUM_LAYERS = 2
D_FFN = 2 * D

NC = 2
NS = 16
NW = NC * NS
PLANES = (L + NW - 1) // NW
NEG = np.float32(-3.0e38)

BLK_P = 400
BLK_E = 512
BLK_L = 2000


def _ln(x):
    m = jnp.mean(x, axis=-1, keepdims=True)
    v = jnp.mean((x - m) ** 2, axis=-1, keepdims=True)
    return (x - m) / jnp.sqrt(v + 1e-5)


def _rln(x):
    return jax.nn.relu(_ln(x))


def _cspec(shape):
    return pl.BlockSpec(shape, lambda i, _s=shape: tuple(0 for _ in _s))


def _bspec(blk, width):
    return pl.BlockSpec((blk, width), lambda i: (i, 0))


def _row_call(body, n_rows, blk, in_widths, out_widths, consts):
    grid = (n_rows // blk,)
    in_specs = [_bspec(blk, w) for w in in_widths]
    in_specs += [_cspec(c) for c in consts]
    out_specs = [_bspec(blk, w) for w in out_widths]
    out_shape = [jax.ShapeDtypeStruct((n_rows, w), jnp.float32)
                 for w in out_widths]
    if len(out_widths) == 1:
        out_specs, out_shape = out_specs[0], out_shape[0]
    return pl.pallas_call(body, grid=grid, in_specs=in_specs,
                          out_specs=out_specs, out_shape=out_shape)


_SC_MESH = plsc.VectorSubcoreMesh(core_axis_name="c", subcore_axis_name="s")


def _wid():
    return lax.axis_index("s") * NC + lax.axis_index("c")


def sc_gather(table, idx, n_out, width, chunk=400):
    nch = n_out // (NW * chunk)

    @functools.partial(
        pl.kernel,
        out_type=jax.ShapeDtypeStruct((n_out, width), jnp.float32),
        mesh=_SC_MESH,
        scratch_types=[
            pltpu.VMEM((chunk,), jnp.int32),
            pltpu.VMEM((chunk,), jnp.int32),
            pltpu.VMEM((chunk, width), jnp.float32),
            pltpu.VMEM((chunk, width), jnp.float32),
            pltpu.SemaphoreType.DMA((6,)),
        ],
    )
    def k(table_hbm, idx_hbm, out_hbm, idx_v0, idx_v1, rows_v0, rows_v1,
          sems):
        w = _wid()
        ib = [idx_v0, idx_v1]
        rb = [rows_v0, rows_v1]
        si = [sems.at[0], sems.at[1]]
        sg = [sems.at[2], sems.at[3]]
        so = [sems.at[4], sems.at[5]]

        def base(c):
            return (w * nch + c) * chunk

        def issue_idx(c):
            return pltpu.async_copy(idx_hbm.at[pl.ds(base(c), chunk)],
                                    ib[c % 2], si[c % 2])

        def issue_gather(c):
            return pltpu.async_copy(table_hbm.at[ib[c % 2]], rb[c % 2],
                                    sg[c % 2])

        def issue_out(c):
            return pltpu.async_copy(rb[c % 2],
                                    out_hbm.at[pl.ds(base(c), chunk)],
                                    so[c % 2])

        hi, hg, ho = {}, {}, {}
        hi[0] = issue_idx(0)
        if nch > 1:
            hi[1] = issue_idx(1)
        hi[0].wait()
        hg[0] = issue_gather(0)
        for c in range(nch):
            if c + 1 < nch:
                hi[c + 1].wait()
                if c >= 1:
                    ho[c - 1].wait()
                hg[c + 1] = issue_gather(c + 1)
            hg[c].wait()
            ho[c] = issue_out(c)
            if c + 2 < nch:
                hi[c + 2] = issue_idx(c + 2)
        if nch > 1:
            ho[nch - 2].wait()
        ho[nch - 1].wait()

    return k(table, idx)


def sc_scatter(rows, idx, n_out, width, chunk=400):
    n_in = rows.shape[0]
    nch = n_in // (NW * chunk)

    @functools.partial(
        pl.kernel,
        out_type=jax.ShapeDtypeStruct((n_out, width), jnp.float32),
        mesh=_SC_MESH,
        scratch_types=[
            pltpu.VMEM((chunk,), jnp.int32),
            pltpu.VMEM((chunk,), jnp.int32),
            pltpu.VMEM((chunk, width), jnp.float32),
            pltpu.VMEM((chunk, width), jnp.float32),
            pltpu.SemaphoreType.DMA((6,)),
        ],
    )
    def k(rows_hbm, idx_hbm, out_hbm, idx_v0, idx_v1, rows_v0, rows_v1,
          sems):
        w = _wid()
        ib = [idx_v0, idx_v1]
        rb = [rows_v0, rows_v1]
        si = [sems.at[0], sems.at[1]]
        sr = [sems.at[2], sems.at[3]]
        so = [sems.at[4], sems.at[5]]

        def base(c):
            return (w * nch + c) * chunk

        def issue_in(c):
            return (pltpu.async_copy(idx_hbm.at[pl.ds(base(c), chunk)],
                                     ib[c % 2], si[c % 2]),
                    pltpu.async_copy(rows_hbm.at[pl.ds(base(c), chunk)],
                                     rb[c % 2], sr[c % 2]))

        hi, ho = {}, {}
        hi[0] = issue_in(0)
        for c in range(nch):
            if c + 1 < nch:
                if c >= 1:
                    ho[c - 1].wait()
                hi[c + 1] = issue_in(c + 1)
            hi[c][0].wait()
            hi[c][1].wait()
            ho[c] = pltpu.async_copy(rb[c % 2], out_hbm.at[ib[c % 2]],
                                     so[c % 2])
        if nch > 1:
            ho[nch - 2].wait()
        ho[nch - 1].wait()

    return k(rows, idx)


def sc_seg_reduce(x, seg_pad, starts, n_rows, width, is_max, expand,
                  chunk=128):
    nfb = width // 16
    out_types = [jax.ShapeDtypeStruct((NW * PLANES * width,), jnp.float32)]
    if expand:
        out_types.append(jax.ShapeDtypeStruct((n_rows + 8, width),
                                              jnp.float32))
    ident = NEG if is_max else np.float32(0.0)
    tab_words = (PLANES + 2) * width

    @functools.partial(
        pl.kernel,
        out_type=tuple(out_types) if expand else out_types[0],
        mesh=_SC_MESH,
        scratch_types=[
            pltpu.VMEM((tab_words,), jnp.float32),
            pltpu.VMEM((chunk * width,), jnp.float32),
            pltpu.VMEM((chunk, width), jnp.float32),
            pltpu.VMEM((chunk + 32,), jnp.int32),
            pltpu.VMEM((chunk,), jnp.int32),
            pltpu.VMEM((NW + 16,), jnp.int32),
            pltpu.SemaphoreType.DMA,
        ],
    )
    def k(x_hbm, seg_hbm, starts_hbm, *rest):
        if expand:
            mx_hbm, y_hbm, tab, buf, obuf, segv, yidx, startv, sem = rest
        else:
            mx_hbm, tab, buf, obuf, segv, yidx, startv, sem = rest
        w = _wid()
        pltpu.sync_copy(starts_hbm, startv)
        my_start = startv[pl.ds(w, 16)][0]
        my_end = startv[pl.ds(w + 1, 16)][0]
        lane0 = w * PLANES
        nchunks = lax.div(my_end - my_start + chunk - 1, chunk)

        def init_body(j, _):
            tab[pl.ds(j * 16, 16)] = jnp.full((16,), ident, jnp.float32)
            return 0
        lax.fori_loop(0, tab_words // 16, init_body, 0)

        def load_seg(cc):
            base_u = my_start + cc * chunk
            base = jnp.maximum(jnp.minimum(base_u, my_end - chunk), 0)
            abase = (base // 8) * 8
            pltpu.sync_copy(seg_hbm.at[pl.ds(abase, chunk + 16)],
                            segv.at[pl.ds(0, chunk + 16)])
            return base_u, base, abase

        def _comb(a, b):
            return jnp.maximum(a, b) if is_max else a + b

        def reduce_chunk(cc, _):
            base_u, base, abase = load_seg(cc)
            pltpu.sync_copy(x_hbm.at[pl.ds(base * width, chunk * width)], buf)

            def row(i, carry):
                cur = carry[0]
                acc = carry[1:]
                g = base + i
                sl = segv[pl.ds(i + (base - abase), 16)][0] - lane0
                valid = jnp.logical_and(g >= base_u, g < my_end)
                sl = jnp.where(valid, sl, PLANES)
                changed = sl != cur

                def flush(_):
                    for f in range(nfb):
                        t = tab[pl.ds(cur * width + f * 16, 16)]
                        tab[pl.ds(cur * width + f * 16, 16)] = _comb(t, acc[f])
                    return 0
                lax.cond(changed, flush, lambda _: 0, 0)
                acc3 = tuple(
                    _comb(jnp.where(changed, ident, acc[f]),
                          buf[pl.ds(i * width + f * 16, 16)])
                    for f in range(nfb))
                return (sl,) + acc3

            iacc = tuple(jnp.full((16,), ident, jnp.float32)
                         for _ in range(nfb))
            fin = lax.fori_loop(0, chunk, row, (PLANES + 1,) + iacc,
                                unroll=4)
            cur = fin[0]
            for f in range(nfb):
                t = tab[pl.ds(cur * width + f * 16, 16)]
                tab[pl.ds(cur * width + f * 16, 16)] = _comb(t, fin[1 + f])
            return 0
        lax.fori_loop(0, nchunks, reduce_chunk, 0)

        if is_max:
            def fix_body(j, _):
                t = tab[pl.ds(j * 16, 16)]
                tab[pl.ds(j * 16, 16)] = jnp.where(t <= NEG, 0.0, t)
                return 0
            lax.fori_loop(0, tab_words // 16, fix_body, 0)

        pltpu.sync_copy(tab.at[pl.ds(0, PLANES * width)],
                        mx_hbm.at[pl.ds(lane0 * width, PLANES * width)])

        if expand:
            def expand_chunk(cc, _):
                base_u, base, abase = load_seg(cc)
                for b in range(chunk // 16):
                    giota = base + b * 16 + lax.iota(jnp.int32, 16)
                    okv = jnp.logical_and(giota >= base_u, giota < my_end)
                    yidx[pl.ds(b * 16, 16)] = jnp.where(okv, giota, n_rows)

                def row(i, _c):
                    sl = jnp.clip(segv[pl.ds(i + (base - abase), 16)][0]
                                  - lane0, 0, PLANES)
                    for f in range(nfb):
                        obuf[i, pl.ds(f * 16, 16)] = tab[
                            pl.ds(sl * width + f * 16, 16)]
                    return _c

                lax.fori_loop(0, chunk, row, 0, unroll=4)
                pltpu.async_copy(obuf, y_hbm.at[yidx], sem).wait()
                return 0
            lax.fori_loop(0, nchunks, expand_chunk, 0)

    outs = k(x.reshape(-1), seg_pad, starts)
    if expand:
        mx, y = outs
        return mx.reshape(NW * PLANES, width)[:L], y[:n_rows]
    return outs.reshape(NW * PLANES, width)[:L]


def sc_attn_aggregate(lg, v, seg_pad, starts, chunk=128):
    tabw = (PLANES + 2) * 16
    taga = (PLANES + 2) * D

    @functools.partial(
        pl.kernel,
        out_type=jax.ShapeDtypeStruct((NW * PLANES * D,), jnp.float32),
        mesh=_SC_MESH,
        scratch_types=[
            pltpu.VMEM((tabw,), jnp.float32),
            pltpu.VMEM((tabw,), jnp.float32),
            pltpu.VMEM((taga,), jnp.float32),
            pltpu.VMEM((chunk * 16,), jnp.float32),
            pltpu.VMEM((chunk * D,), jnp.float32),
            pltpu.VMEM((chunk + 32,), jnp.int32),
            pltpu.VMEM((NW + 16,), jnp.int32),
        ],
    )
    def k(lg_hbm, v_hbm, seg_hbm, starts_hbm, out_hbm,
          mtab, stab, atab, lbuf, vbuf, segv, startv):
        w = _wid()
        pltpu.sync_copy(starts_hbm, startv)
        my_start = startv[pl.ds(w, 16)][0]
        my_end = startv[pl.ds(w + 1, 16)][0]
        lane0 = w * PLANES
        nchunks = lax.div(my_end - my_start + chunk - 1, chunk)

        def init16(j, _):
            mtab[pl.ds(j * 16, 16)] = jnp.full((16,), NEG, jnp.float32)
            stab[pl.ds(j * 16, 16)] = jnp.zeros((16,), jnp.float32)
            return 0
        lax.fori_loop(0, tabw // 16, init16, 0)

        def inita(j, _):
            atab[pl.ds(j * 16, 16)] = jnp.zeros((16,), jnp.float32)
            return 0
        lax.fori_loop(0, taga // 16, inita, 0)

        def load_seg(cc):
            base_u = my_start + cc * chunk
            base = jnp.maximum(jnp.minimum(base_u, my_end - chunk), 0)
            abase = (base // 8) * 8
            pltpu.sync_copy(seg_hbm.at[pl.ds(abase, chunk + 16)],
                            segv.at[pl.ds(0, chunk + 16)])
            return base_u, base, abase

        def rowmeta(base_u, base, abase, i):
            g = base + i
            sl = segv[pl.ds(i + (base - abase), 16)][0] - lane0
            valid = jnp.logical_and(g >= base_u, g < my_end)
            return jnp.where(valid, sl, PLANES)

        def pass_mx(cc, _):
            base_u, base, abase = load_seg(cc)
            pltpu.sync_copy(lg_hbm.at[pl.ds(base * 16, chunk * 16)], lbuf)

            def row(i, carry):
                cur, acc = carry
                sl = rowmeta(base_u, base, abase, i)
                changed = sl != cur

                def flush(_):
                    t = mtab[pl.ds(cur * 16, 16)]
                    mtab[pl.ds(cur * 16, 16)] = jnp.maximum(t, acc)
                    return 0
                lax.cond(changed, flush, lambda _: 0, 0)
                acc2 = jnp.where(changed, NEG, acc)
                return (sl, jnp.maximum(acc2, lbuf[pl.ds(i * 16, 16)]))

            cur, acc = lax.fori_loop(
                0, chunk, row,
                (PLANES + 1, jnp.full((16,), NEG, jnp.float32)), unroll=4)
            t = mtab[pl.ds(cur * 16, 16)]
            mtab[pl.ds(cur * 16, 16)] = jnp.maximum(t, acc)
            return 0
        lax.fori_loop(0, nchunks, pass_mx, 0)

        def pass_sum(cc, _):
            base_u, base, abase = load_seg(cc)
            pltpu.sync_copy(lg_hbm.at[pl.ds(base * 16, chunk * 16)], lbuf)

            def row(i, carry):
                cur, acc = carry
                sl = rowmeta(base_u, base, abase, i)
                changed = sl != cur

                def flush(_):
                    t = stab[pl.ds(cur * 16, 16)]
                    stab[pl.ds(cur * 16, 16)] = t + acc
                    return 0
                lax.cond(changed, flush, lambda _: 0, 0)
                acc2 = jnp.where(changed, 0.0, acc)
                mv = mtab[pl.ds(sl * 16, 16)]
                ev = jnp.exp(lbuf[pl.ds(i * 16, 16)] - mv)
                return (sl, acc2 + ev)

            cur, acc = lax.fori_loop(
                0, chunk, row,
                (PLANES + 1, jnp.zeros((16,), jnp.float32)), unroll=4)
            t = stab[pl.ds(cur * 16, 16)]
            stab[pl.ds(cur * 16, 16)] = t + acc
            return 0
        lax.fori_loop(0, nchunks, pass_sum, 0)

        def recip(j, _):
            t = stab[pl.ds(j * 16, 16)]
            stab[pl.ds(j * 16, 16)] = 1.0 / (t + 1e-16)
            return 0
        lax.fori_loop(0, tabw // 16, recip, 0)

        def pass_wv(cc, _):
            base_u, base, abase = load_seg(cc)
            pltpu.sync_copy(lg_hbm.at[pl.ds(base * 16, chunk * 16)], lbuf)
            pltpu.sync_copy(v_hbm.at[pl.ds(base * D, chunk * D)], vbuf)

            def row(i, carry):
                cur = carry[0]
                acc = carry[1:]
                sl = rowmeta(base_u, base, abase, i)
                changed = sl != cur

                def flush(_):
                    for h in range(H):
                        a = atab[pl.ds(cur * D + h * 16, 16)]
                        atab[pl.ds(cur * D + h * 16, 16)] = a + acc[h]
                    return 0
                lax.cond(changed, flush, lambda _: 0, 0)
                mv = mtab[pl.ds(sl * 16, 16)]
                sv = stab[pl.ds(sl * 16, 16)]
                wt = jnp.exp(lbuf[pl.ds(i * 16, 16)] - mv) * sv
                acc3 = tuple(
                    jnp.where(changed, 0.0, acc[h])
                    + vbuf[pl.ds(i * D + h * 16, 16)] * wt[h]
                    for h in range(H))
                return (sl,) + acc3

            zero = jnp.zeros((16,), jnp.float32)
            fin = lax.fori_loop(
                0, chunk, row,
                (PLANES + 1,) + tuple(zero for _ in range(H)), unroll=4)
            cur = fin[0]
            for h in range(H):
                a = atab[pl.ds(cur * D + h * 16, 16)]
                atab[pl.ds(cur * D + h * 16, 16)] = a + fin[1 + h]
            return 0
        lax.fori_loop(0, nchunks, pass_wv, 0)

        pltpu.sync_copy(atab.at[pl.ds(0, PLANES * D)],
                        out_hbm.at[pl.ds(lane0 * D, PLANES * D)])

    out = k(lg.reshape(-1), v.reshape(-1), seg_pad, starts)
    return out.reshape(NW * PLANES, D)[:L]


def _points1_body(nf, projW, projB, f1aW, f1aB, f1bW, f1bB, x0o, x1o):
    x0 = _rln(jnp.dot(nf[...], projW[...]) + projB[...])
    t = _rln(jnp.dot(x0, f1aW[...]) + f1aB[...])
    x1 = _rln(jnp.dot(t, f1bW[...]) + f1bB[...])
    x0o[...] = x0
    x1o[...] = x1


def _fc2_body(xin, x1, y1, aWt, aWb, aB, bW, bB, f1aW, f1aB, f1bW, f1bB,
              outo, x2o):
    h = _rln(jnp.dot(x1[...], aWt[...]) + jnp.dot(y1[...], aWb[...]) + aB[...])
    h = _rln(jnp.dot(h, bW[...]) + bB[...])
    out = _ln(xin[...] + h)
    t = _rln(jnp.dot(out, f1aW[...]) + f1aB[...])
    x2 = _rln(jnp.dot(t, f1bW[...]) + f1bB[...])
    outo[...] = out
    x2o[...] = x2


def _fc2_final_body(xin, x1, y1, aWt, aWb, aB, bW, bB, outo):
    h = _rln(jnp.dot(x1[...], aWt[...]) + jnp.dot(y1[...], aWb[...]) + aB[...])
    h = _rln(jnp.dot(h, bW[...]) + bB[...])
    outo[...] = _ln(xin[...] + h)


def _rpe_body(rp, rW, rB, eao):
    eao[...] = _rln(jnp.dot(rp[...], rW[...]) + rB[...])


def _edge1_body(xd, xs, ea, W1, W2, W3, mB, euW, euB, qW, kW, vW, hsel,
                eao, vo, lo):
    mem = _rln(jnp.dot(xd[...], W1[...]) + jnp.dot(xs[...], W2[...])
               + jnp.dot(ea[...], W3[...]) + mB[...])
    delta = _rln(jnp.dot(mem, euW[...]) + euB[...])
    eao[...] = _ln(ea[...] + delta)
    q = jnp.dot(xd[...], qW[...])
    kk = jnp.dot(mem, kW[...])
    vo[...] = jnp.dot(mem, vW[...])
    lo[...] = jnp.dot(q * kk, hsel[...]) * (1.0 / np.sqrt(DH))


def _lane_body(x, aggr, oW, f1W, f1B, f2W, f2B, xo):
    x1 = _ln(x[...] + jnp.dot(aggr[...], oW[...]))
    h = jax.nn.relu(jnp.dot(x1, f1W[...]) + f1B[...])
    xo[...] = _ln(x1 + jnp.dot(h, f2W[...]) + f2B[...])


def kernel(node_feats, nodes_of_lanes, l2l_edges, l2l_fused_rpes, params):
    p = params
    f32 = jnp.float32

    src, dst = l2l_edges[0], l2l_edges[1]
    perm = jnp.argsort(dst).astype(jnp.int32)
    dst_s = dst[perm]
    src_s = src[perm]
    lane_cuts = jnp.arange(NW + 16, dtype=jnp.int32) * PLANES
    e_starts = jnp.minimum(
        jnp.searchsorted(dst_s, lane_cuts), E).astype(jnp.int32)
    n_starts = jnp.minimum(
        jnp.searchsorted(nodes_of_lanes, lane_cuts), NPTS).astype(jnp.int32)
    nol_pad = jnp.concatenate([nodes_of_lanes, jnp.full((16,), L, jnp.int32)])
    dst_pad = jnp.concatenate([dst_s, jnp.full((16,), L, jnp.int32)])

    nf_pad = jnp.zeros((NPTS, D), f32).at[:, :IN_DIM].set(node_feats)
    projW = jnp.zeros((D, D), f32).at[:IN_DIM].set(p['proj_W'])

    shp_dd = (D, D)
    x0, x1 = _row_call(
        _points1_body, NPTS, BLK_P, [D], [D, D],
        consts=(shp_dd, (D,), shp_dd, (D,), shp_dd, (D,)),
    )(nf_pad, projW, p['proj_b'], p['pa_fc1a_W'], p['pa_fc1a_b'],
      p['pa_fc1b_W'], p['pa_fc1b_b'])

    _, y1 = sc_seg_reduce(x1, nol_pad, n_starts, NPTS, D,
                          is_max=True, expand=True)

    out_pa, x2 = _row_call(
        _fc2_body, NPTS, BLK_P, [D, D, D], [D, D],
        consts=(shp_dd, shp_dd, (D,), shp_dd, (D,), shp_dd, (D,),
                shp_dd, (D,)),
    )(x0, x1, y1, p['pa_fc2a_W'][:D], p['pa_fc2a_W'][D:], p['pa_fc2a_b'],
      p['pa_fc2b_W'], p['pa_fc2b_b'],
      p['la_fc1a_W'], p['la_fc1a_b'], p['la_fc1b_W'], p['la_fc1b_b'])

    _, y2 = sc_seg_reduce(x2, nol_pad, n_starts, NPTS, D,
                          is_max=True, expand=True)

    out_la = _row_call(
        _fc2_final_body, NPTS, BLK_P, [D, D, D], [D],
        consts=(shp_dd, shp_dd, (D,), shp_dd, (D,)),
    )(out_pa, x2, y2, p['la_fc2a_W'][:D], p['la_fc2a_W'][D:],
      p['la_fc2a_b'], p['la_fc2b_W'], p['la_fc2b_b'])

    x = sc_seg_reduce(out_la, nol_pad, n_starts, NPTS, D,
                      is_max=True, expand=False)

    rpW = jnp.zeros((16, D), f32).at[:D_RPE].set(p['rpe_W'])
    rpes_p = jnp.zeros((E, 16), f32).at[:, :D_RPE].set(l2l_fused_rpes)
    ea0 = _row_call(
        _rpe_body, E, BLK_E, [16], [D],
        consts=((16, D), (D,)),
    )(rpes_p, rpW, p['rpe_b'])
    ea = sc_gather(ea0, perm, E, D)

    hsel_np = np.zeros((D, 16), np.float32)
    for h in range(H):
        hsel_np[h * DH:(h + 1) * DH, h] = 1.0
    hsel = jnp.asarray(hsel_np)

    for l in range(NUM_LAYERS):
        pre = 'l%d_' % l
        mW = p[pre + 'mem_W']
        xd = sc_gather(x, dst_s, E, D)
        xs = sc_gather(x, src_s, E, D)
        ea, v, lg = pl.pallas_call(
            _edge1_body,
            grid=(E // BLK_E,),
            in_specs=[_bspec(BLK_E, D)] * 3 + [
                _cspec(shp_dd), _cspec(shp_dd), _cspec(shp_dd), _cspec((D,)),
                _cspec(shp_dd), _cspec((D,)), _cspec(shp_dd), _cspec(shp_dd),
                _cspec(shp_dd), _cspec((D, 16)),
            ],
            out_specs=[_bspec(BLK_E, D), _bspec(BLK_E, D), _bspec(BLK_E, 16)],
            out_shape=[jax.ShapeDtypeStruct((E, D), f32),
                       jax.ShapeDtypeStruct((E, D), f32),
                       jax.ShapeDtypeStruct((E, 16), f32)],
        )(xd, xs, ea, mW[:D], mW[D:2 * D], mW[2 * D:], p[pre + 'mem_b'],
          p[pre + 'eu_W'], p[pre + 'eu_b'], p[pre + 'q_W'], p[pre + 'k_W'],
          p[pre + 'v_W'], hsel)

        aggr = sc_attn_aggregate(lg, v, dst_pad, e_starts)

        x = _row_call(
            _lane_body, L, BLK_L, [D, D], [D],
            consts=(shp_dd, (D, D_FFN), (D_FFN,), (D_FFN, D), (D,)),
        )(x, aggr, p[pre + 'o_W'], p[pre + 'ffn1_W'], p[pre + 'ffn1_b'],
          p[pre + 'ffn2_W'], p[pre + 'ffn2_b'])

    l2l_attr = sc_scatter(ea, perm, E, D)
    return (x, l2l_attr)


# speedup vs baseline: 1.9911x; 1.0473x over previous
import functools

import jax
import jax.numpy as jnp
import numpy as np
from jax import lax
from jax.experimental import pallas as pl
from jax.experimental.pallas import tpu as pltpu
from jax.experimental.pallas import tpu_sc as plsc

D = 128
H = 8
DH = D // H
L = 10000
NPTS = 100000
E = 320000
IN_DIM = 10
D_RPE = 8
NUM_LAYERS = 2
D_FFN = 2 * D

NC = 2
NS = 16
NW = NC * NS
PLANES = (L + NW - 1) // NW
NEG = np.float32(-3.0e38)

BLK_P = 400
BLK_E = 512
BLK_L = 2000


def _ln(x):
    m = jnp.mean(x, axis=-1, keepdims=True)
    v = jnp.mean((x - m) ** 2, axis=-1, keepdims=True)
    return (x - m) / jnp.sqrt(v + 1e-5)


def _rln(x):
    return jax.nn.relu(_ln(x))


def _cspec(shape):
    return pl.BlockSpec(shape, lambda i, _s=shape: tuple(0 for _ in _s))


def _bspec(blk, width):
    return pl.BlockSpec((blk, width), lambda i: (i, 0))


def _row_call(body, n_rows, blk, in_widths, out_widths, consts):
    grid = (n_rows // blk,)
    in_specs = [_bspec(blk, w) for w in in_widths]
    in_specs += [_cspec(c) for c in consts]
    out_specs = [_bspec(blk, w) for w in out_widths]
    out_shape = [jax.ShapeDtypeStruct((n_rows, w), jnp.float32)
                 for w in out_widths]
    if len(out_widths) == 1:
        out_specs, out_shape = out_specs[0], out_shape[0]
    return pl.pallas_call(body, grid=grid, in_specs=in_specs,
                          out_specs=out_specs, out_shape=out_shape)


_SC_MESH = plsc.VectorSubcoreMesh(core_axis_name="c", subcore_axis_name="s")


def _wid():
    return lax.axis_index("s") * NC + lax.axis_index("c")


def sc_gather(table, idx, n_out, width, chunk=400):
    nch = n_out // (NW * chunk)

    @functools.partial(
        pl.kernel,
        out_type=jax.ShapeDtypeStruct((n_out, width), jnp.float32),
        mesh=_SC_MESH,
        scratch_types=[
            pltpu.VMEM((chunk,), jnp.int32),
            pltpu.VMEM((chunk,), jnp.int32),
            pltpu.VMEM((chunk, width), jnp.float32),
            pltpu.VMEM((chunk, width), jnp.float32),
            pltpu.SemaphoreType.DMA((6,)),
        ],
    )
    def k(table_hbm, idx_hbm, out_hbm, idx_v0, idx_v1, rows_v0, rows_v1,
          sems):
        w = _wid()
        ib = [idx_v0, idx_v1]
        rb = [rows_v0, rows_v1]
        si = [sems.at[0], sems.at[1]]
        sg = [sems.at[2], sems.at[3]]
        so = [sems.at[4], sems.at[5]]

        def base(c):
            return (w * nch + c) * chunk

        def issue_idx(c):
            return pltpu.async_copy(idx_hbm.at[pl.ds(base(c), chunk)],
                                    ib[c % 2], si[c % 2])

        def issue_gather(c):
            return pltpu.async_copy(table_hbm.at[ib[c % 2]], rb[c % 2],
                                    sg[c % 2])

        def issue_out(c):
            return pltpu.async_copy(rb[c % 2],
                                    out_hbm.at[pl.ds(base(c), chunk)],
                                    so[c % 2])

        hi, hg, ho = {}, {}, {}
        hi[0] = issue_idx(0)
        if nch > 1:
            hi[1] = issue_idx(1)
        hi[0].wait()
        hg[0] = issue_gather(0)
        for c in range(nch):
            if c + 1 < nch:
                hi[c + 1].wait()
                if c >= 1:
                    ho[c - 1].wait()
                hg[c + 1] = issue_gather(c + 1)
            hg[c].wait()
            ho[c] = issue_out(c)
            if c + 2 < nch:
                hi[c + 2] = issue_idx(c + 2)
        if nch > 1:
            ho[nch - 2].wait()
        ho[nch - 1].wait()

    return k(table, idx)


def sc_scatter(rows, idx, n_out, width, chunk=400):
    n_in = rows.shape[0]
    nch = n_in // (NW * chunk)

    @functools.partial(
        pl.kernel,
        out_type=jax.ShapeDtypeStruct((n_out, width), jnp.float32),
        mesh=_SC_MESH,
        scratch_types=[
            pltpu.VMEM((chunk,), jnp.int32),
            pltpu.VMEM((chunk,), jnp.int32),
            pltpu.VMEM((chunk, width), jnp.float32),
            pltpu.VMEM((chunk, width), jnp.float32),
            pltpu.SemaphoreType.DMA((6,)),
        ],
    )
    def k(rows_hbm, idx_hbm, out_hbm, idx_v0, idx_v1, rows_v0, rows_v1,
          sems):
        w = _wid()
        ib = [idx_v0, idx_v1]
        rb = [rows_v0, rows_v1]
        si = [sems.at[0], sems.at[1]]
        sr = [sems.at[2], sems.at[3]]
        so = [sems.at[4], sems.at[5]]

        def base(c):
            return (w * nch + c) * chunk

        def issue_in(c):
            return (pltpu.async_copy(idx_hbm.at[pl.ds(base(c), chunk)],
                                     ib[c % 2], si[c % 2]),
                    pltpu.async_copy(rows_hbm.at[pl.ds(base(c), chunk)],
                                     rb[c % 2], sr[c % 2]))

        hi, ho = {}, {}
        hi[0] = issue_in(0)
        for c in range(nch):
            if c + 1 < nch:
                if c >= 1:
                    ho[c - 1].wait()
                hi[c + 1] = issue_in(c + 1)
            hi[c][0].wait()
            hi[c][1].wait()
            ho[c] = pltpu.async_copy(rb[c % 2], out_hbm.at[ib[c % 2]],
                                     so[c % 2])
        if nch > 1:
            ho[nch - 2].wait()
        ho[nch - 1].wait()

    return k(rows, idx)


def sc_seg_reduce(x, seg_pad, starts, n_rows, width, is_max, expand,
                  chunk=128):
    nfb = width // 16
    out_types = [jax.ShapeDtypeStruct((NW * PLANES * width,), jnp.float32)]
    if expand:
        out_types.append(jax.ShapeDtypeStruct((n_rows + 8, width),
                                              jnp.float32))
    ident = NEG if is_max else np.float32(0.0)
    tab_words = (PLANES + 2) * width

    @functools.partial(
        pl.kernel,
        out_type=tuple(out_types) if expand else out_types[0],
        mesh=_SC_MESH,
        scratch_types=[
            pltpu.VMEM((tab_words,), jnp.float32),
            pltpu.VMEM((chunk * width,), jnp.float32),
            pltpu.VMEM((chunk, width), jnp.float32),
            pltpu.VMEM((chunk + 32,), jnp.int32),
            pltpu.VMEM((chunk,), jnp.int32),
            pltpu.VMEM((NW + 16,), jnp.int32),
            pltpu.SemaphoreType.DMA,
        ],
    )
    def k(x_hbm, seg_hbm, starts_hbm, *rest):
        if expand:
            mx_hbm, y_hbm, tab, buf, obuf, segv, yidx, startv, sem = rest
        else:
            mx_hbm, tab, buf, obuf, segv, yidx, startv, sem = rest
        w = _wid()
        pltpu.sync_copy(starts_hbm, startv)
        my_start = startv[pl.ds(w, 16)][0]
        my_end = startv[pl.ds(w + 1, 16)][0]
        lane0 = w * PLANES
        nchunks = lax.div(my_end - my_start + chunk - 1, chunk)

        def init_body(j, _):
            tab[pl.ds(j * 16, 16)] = jnp.full((16,), ident, jnp.float32)
            return 0
        lax.fori_loop(0, tab_words // 16, init_body, 0)

        def load_seg(cc):
            base_u = my_start + cc * chunk
            base = jnp.maximum(jnp.minimum(base_u, my_end - chunk), 0)
            abase = (base // 8) * 8
            pltpu.sync_copy(seg_hbm.at[pl.ds(abase, chunk + 16)],
                            segv.at[pl.ds(0, chunk + 16)])
            return base_u, base, abase

        def _comb(a, b):
            return jnp.maximum(a, b) if is_max else a + b

        def reduce_chunk(cc, _):
            base_u, base, abase = load_seg(cc)
            pltpu.sync_copy(x_hbm.at[pl.ds(base * width, chunk * width)], buf)

            def row(i, carry):
                cur = carry[0]
                acc = carry[1:]
                g = base + i
                sl = segv[pl.ds(i + (base - abase), 16)][0] - lane0
                valid = jnp.logical_and(g >= base_u, g < my_end)
                sl = jnp.where(valid, sl, PLANES)
                changed = sl != cur

                def flush(_):
                    for f in range(nfb):
                        t = tab[pl.ds(cur * width + f * 16, 16)]
                        tab[pl.ds(cur * width + f * 16, 16)] = _comb(t, acc[f])
                    return 0
                lax.cond(changed, flush, lambda _: 0, 0)
                acc3 = tuple(
                    _comb(jnp.where(changed, ident, acc[f]),
                          buf[pl.ds(i * width + f * 16, 16)])
                    for f in range(nfb))
                return (sl,) + acc3

            iacc = tuple(jnp.full((16,), ident, jnp.float32)
                         for _ in range(nfb))
            fin = lax.fori_loop(0, chunk, row, (PLANES + 1,) + iacc,
                                unroll=4)
            cur = fin[0]
            for f in range(nfb):
                t = tab[pl.ds(cur * width + f * 16, 16)]
                tab[pl.ds(cur * width + f * 16, 16)] = _comb(t, fin[1 + f])
            return 0
        lax.fori_loop(0, nchunks, reduce_chunk, 0)

        if is_max:
            def fix_body(j, _):
                t = tab[pl.ds(j * 16, 16)]
                tab[pl.ds(j * 16, 16)] = jnp.where(t <= NEG, 0.0, t)
                return 0
            lax.fori_loop(0, tab_words // 16, fix_body, 0)

        pltpu.sync_copy(tab.at[pl.ds(0, PLANES * width)],
                        mx_hbm.at[pl.ds(lane0 * width, PLANES * width)])

        if expand:
            def expand_chunk(cc, _):
                base_u, base, abase = load_seg(cc)
                for b in range(chunk // 16):
                    giota = base + b * 16 + lax.iota(jnp.int32, 16)
                    okv = jnp.logical_and(giota >= base_u, giota < my_end)
                    yidx[pl.ds(b * 16, 16)] = jnp.where(okv, giota, n_rows)

                def row(i, _c):
                    sl = jnp.clip(segv[pl.ds(i + (base - abase), 16)][0]
                                  - lane0, 0, PLANES)
                    for f in range(nfb):
                        obuf[i, pl.ds(f * 16, 16)] = tab[
                            pl.ds(sl * width + f * 16, 16)]
                    return _c

                lax.fori_loop(0, chunk, row, 0, unroll=4)
                pltpu.async_copy(obuf, y_hbm.at[yidx], sem).wait()
                return 0
            lax.fori_loop(0, nchunks, expand_chunk, 0)

    outs = k(x.reshape(-1), seg_pad, starts)
    if expand:
        mx, y = outs
        return mx.reshape(NW * PLANES, width)[:L], y[:n_rows]
    return outs.reshape(NW * PLANES, width)[:L]


def sc_attn_aggregate(lg, v, seg_pad, starts, chunk=128):
    tabw = (PLANES + 2) * 16
    taga = (PLANES + 2) * D

    @functools.partial(
        pl.kernel,
        out_type=jax.ShapeDtypeStruct((NW * PLANES * D,), jnp.float32),
        mesh=_SC_MESH,
        scratch_types=[
            pltpu.VMEM((tabw,), jnp.float32),
            pltpu.VMEM((tabw,), jnp.float32),
            pltpu.VMEM((taga,), jnp.float32),
            pltpu.VMEM((chunk * 16,), jnp.float32),
            pltpu.VMEM((chunk * D,), jnp.float32),
            pltpu.VMEM((chunk + 32,), jnp.int32),
            pltpu.VMEM((NW + 16,), jnp.int32),
        ],
    )
    def k(lg_hbm, v_hbm, seg_hbm, starts_hbm, out_hbm,
          mtab, stab, atab, lbuf, vbuf, segv, startv):
        w = _wid()
        pltpu.sync_copy(starts_hbm, startv)
        my_start = startv[pl.ds(w, 16)][0]
        my_end = startv[pl.ds(w + 1, 16)][0]
        lane0 = w * PLANES
        nchunks = lax.div(my_end - my_start + chunk - 1, chunk)

        def init16(j, _):
            mtab[pl.ds(j * 16, 16)] = jnp.full((16,), NEG, jnp.float32)
            stab[pl.ds(j * 16, 16)] = jnp.zeros((16,), jnp.float32)
            return 0
        lax.fori_loop(0, tabw // 16, init16, 0)

        def inita(j, _):
            atab[pl.ds(j * 16, 16)] = jnp.zeros((16,), jnp.float32)
            return 0
        lax.fori_loop(0, taga // 16, inita, 0)

        def load_seg(cc):
            base_u = my_start + cc * chunk
            base = jnp.maximum(jnp.minimum(base_u, my_end - chunk), 0)
            abase = (base // 8) * 8
            pltpu.sync_copy(seg_hbm.at[pl.ds(abase, chunk + 16)],
                            segv.at[pl.ds(0, chunk + 16)])
            return base_u, base, abase

        def rowmeta(base_u, base, abase, i):
            g = base + i
            sl = segv[pl.ds(i + (base - abase), 16)][0] - lane0
            valid = jnp.logical_and(g >= base_u, g < my_end)
            return jnp.where(valid, sl, PLANES)

        def pass_ms(cc, _):
            base_u, base, abase = load_seg(cc)
            pltpu.sync_copy(lg_hbm.at[pl.ds(base * 16, chunk * 16)], lbuf)

            def merge(cur, macc, sacc):
                mo = mtab[pl.ds(cur * 16, 16)]
                so = stab[pl.ds(cur * 16, 16)]
                mm = jnp.maximum(mo, macc)
                stab[pl.ds(cur * 16, 16)] = (so * jnp.exp(mo - mm)
                                             + sacc * jnp.exp(macc - mm))
                mtab[pl.ds(cur * 16, 16)] = mm

            def row(i, carry):
                cur, macc, sacc = carry
                sl = rowmeta(base_u, base, abase, i)
                changed = sl != cur

                def flush(_):
                    merge(cur, macc, sacc)
                    return 0
                lax.cond(changed, flush, lambda _: 0, 0)
                m2 = jnp.where(changed, NEG, macc)
                s2 = jnp.where(changed, 0.0, sacc)
                lv = lbuf[pl.ds(i * 16, 16)]
                mn = jnp.maximum(m2, lv)
                s3 = s2 * jnp.exp(m2 - mn) + jnp.exp(lv - mn)
                return (sl, mn, s3)

            cur, macc, sacc = lax.fori_loop(
                0, chunk, row,
                (PLANES + 1, jnp.full((16,), NEG, jnp.float32),
                 jnp.zeros((16,), jnp.float32)), unroll=4)
            merge(cur, macc, sacc)
            return 0
        lax.fori_loop(0, nchunks, pass_ms, 0)

        def recip(j, _):
            t = stab[pl.ds(j * 16, 16)]
            stab[pl.ds(j * 16, 16)] = 1.0 / (t + 1e-16)
            return 0
        lax.fori_loop(0, tabw // 16, recip, 0)

        def pass_wv(cc, _):
            base_u, base, abase = load_seg(cc)
            pltpu.sync_copy(lg_hbm.at[pl.ds(base * 16, chunk * 16)], lbuf)
            pltpu.sync_copy(v_hbm.at[pl.ds(base * D, chunk * D)], vbuf)

            def row(i, carry):
                cur = carry[0]
                acc = carry[1:]
                sl = rowmeta(base_u, base, abase, i)
                changed = sl != cur

                def flush(_):
                    for h in range(H):
                        a = atab[pl.ds(cur * D + h * 16, 16)]
                        atab[pl.ds(cur * D + h * 16, 16)] = a + acc[h]
                    return 0
                lax.cond(changed, flush, lambda _: 0, 0)
                mv = mtab[pl.ds(sl * 16, 16)]
                sv = stab[pl.ds(sl * 16, 16)]
                wt = jnp.exp(lbuf[pl.ds(i * 16, 16)] - mv) * sv
                acc3 = tuple(
                    jnp.where(changed, 0.0, acc[h])
                    + vbuf[pl.ds(i * D + h * 16, 16)] * wt[h]
                    for h in range(H))
                return (sl,) + acc3

            zero = jnp.zeros((16,), jnp.float32)
            fin = lax.fori_loop(
                0, chunk, row,
                (PLANES + 1,) + tuple(zero for _ in range(H)), unroll=4)
            cur = fin[0]
            for h in range(H):
                a = atab[pl.ds(cur * D + h * 16, 16)]
                atab[pl.ds(cur * D + h * 16, 16)] = a + fin[1 + h]
            return 0
        lax.fori_loop(0, nchunks, pass_wv, 0)

        pltpu.sync_copy(atab.at[pl.ds(0, PLANES * D)],
                        out_hbm.at[pl.ds(lane0 * D, PLANES * D)])

    out = k(lg.reshape(-1), v.reshape(-1), seg_pad, starts)
    return out.reshape(NW * PLANES, D)[:L]


def _points1_body(nf, projW, projB, f1aW, f1aB, f1bW, f1bB, x0o, x1o):
    x0 = _rln(jnp.dot(nf[...], projW[...]) + projB[...])
    t = _rln(jnp.dot(x0, f1aW[...]) + f1aB[...])
    x1 = _rln(jnp.dot(t, f1bW[...]) + f1bB[...])
    x0o[...] = x0
    x1o[...] = x1


def _fc2_body(xin, x1, y1, aWt, aWb, aB, bW, bB, f1aW, f1aB, f1bW, f1bB,
              outo, x2o):
    h = _rln(jnp.dot(x1[...], aWt[...]) + jnp.dot(y1[...], aWb[...]) + aB[...])
    h = _rln(jnp.dot(h, bW[...]) + bB[...])
    out = _ln(xin[...] + h)
    t = _rln(jnp.dot(out, f1aW[...]) + f1aB[...])
    x2 = _rln(jnp.dot(t, f1bW[...]) + f1bB[...])
    outo[...] = out
    x2o[...] = x2


def _fc2_final_body(xin, x1, y1, aWt, aWb, aB, bW, bB, outo):
    h = _rln(jnp.dot(x1[...], aWt[...]) + jnp.dot(y1[...], aWb[...]) + aB[...])
    h = _rln(jnp.dot(h, bW[...]) + bB[...])
    outo[...] = _ln(xin[...] + h)


def _rpe_body(rp, rW, rB, eao):
    eao[...] = _rln(jnp.dot(rp[...], rW[...]) + rB[...])


def _edge1_body(xd, xs, ea, W1, W2, W3, mB, euW, euB, qW, kW, vW, hsel,
                eao, vo, lo):
    mem = _rln(jnp.dot(xd[...], W1[...]) + jnp.dot(xs[...], W2[...])
               + jnp.dot(ea[...], W3[...]) + mB[...])
    delta = _rln(jnp.dot(mem, euW[...]) + euB[...])
    eao[...] = _ln(ea[...] + delta)
    q = jnp.dot(xd[...], qW[...])
    kk = jnp.dot(mem, kW[...])
    vo[...] = jnp.dot(mem, vW[...])
    lo[...] = jnp.dot(q * kk, hsel[...]) * (1.0 / np.sqrt(DH))


def _lane_body(x, aggr, oW, f1W, f1B, f2W, f2B, xo):
    x1 = _ln(x[...] + jnp.dot(aggr[...], oW[...]))
    h = jax.nn.relu(jnp.dot(x1, f1W[...]) + f1B[...])
    xo[...] = _ln(x1 + jnp.dot(h, f2W[...]) + f2B[...])


def kernel(node_feats, nodes_of_lanes, l2l_edges, l2l_fused_rpes, params):
    p = params
    f32 = jnp.float32

    src, dst = l2l_edges[0], l2l_edges[1]
    perm = jnp.argsort(dst).astype(jnp.int32)
    dst_s = dst[perm]
    src_s = src[perm]
    lane_cuts = jnp.arange(NW + 16, dtype=jnp.int32) * PLANES
    e_starts = jnp.minimum(
        jnp.searchsorted(dst_s, lane_cuts), E).astype(jnp.int32)
    n_starts = jnp.minimum(
        jnp.searchsorted(nodes_of_lanes, lane_cuts), NPTS).astype(jnp.int32)
    nol_pad = jnp.concatenate([nodes_of_lanes, jnp.full((16,), L, jnp.int32)])
    dst_pad = jnp.concatenate([dst_s, jnp.full((16,), L, jnp.int32)])

    nf_pad = jnp.zeros((NPTS, D), f32).at[:, :IN_DIM].set(node_feats)
    projW = jnp.zeros((D, D), f32).at[:IN_DIM].set(p['proj_W'])

    shp_dd = (D, D)
    x0, x1 = _row_call(
        _points1_body, NPTS, BLK_P, [D], [D, D],
        consts=(shp_dd, (D,), shp_dd, (D,), shp_dd, (D,)),
    )(nf_pad, projW, p['proj_b'], p['pa_fc1a_W'], p['pa_fc1a_b'],
      p['pa_fc1b_W'], p['pa_fc1b_b'])

    _, y1 = sc_seg_reduce(x1, nol_pad, n_starts, NPTS, D,
                          is_max=True, expand=True)

    out_pa, x2 = _row_call(
        _fc2_body, NPTS, BLK_P, [D, D, D], [D, D],
        consts=(shp_dd, shp_dd, (D,), shp_dd, (D,), shp_dd, (D,),
                shp_dd, (D,)),
    )(x0, x1, y1, p['pa_fc2a_W'][:D], p['pa_fc2a_W'][D:], p['pa_fc2a_b'],
      p['pa_fc2b_W'], p['pa_fc2b_b'],
      p['la_fc1a_W'], p['la_fc1a_b'], p['la_fc1b_W'], p['la_fc1b_b'])

    _, y2 = sc_seg_reduce(x2, nol_pad, n_starts, NPTS, D,
                          is_max=True, expand=True)

    out_la = _row_call(
        _fc2_final_body, NPTS, BLK_P, [D, D, D], [D],
        consts=(shp_dd, shp_dd, (D,), shp_dd, (D,)),
    )(out_pa, x2, y2, p['la_fc2a_W'][:D], p['la_fc2a_W'][D:],
      p['la_fc2a_b'], p['la_fc2b_W'], p['la_fc2b_b'])

    x = sc_seg_reduce(out_la, nol_pad, n_starts, NPTS, D,
                      is_max=True, expand=False)

    rpW = jnp.zeros((16, D), f32).at[:D_RPE].set(p['rpe_W'])
    rpes_p = jnp.zeros((E, 16), f32).at[:, :D_RPE].set(l2l_fused_rpes)
    ea0 = _row_call(
        _rpe_body, E, BLK_E, [16], [D],
        consts=((16, D), (D,)),
    )(rpes_p, rpW, p['rpe_b'])
    ea = sc_gather(ea0, perm, E, D)

    hsel_np = np.zeros((D, 16), np.float32)
    for h in range(H):
        hsel_np[h * DH:(h + 1) * DH, h] = 1.0
    hsel = jnp.asarray(hsel_np)

    for l in range(NUM_LAYERS):
        pre = 'l%d_' % l
        mW = p[pre + 'mem_W']
        xd = sc_gather(x, dst_s, E, D)
        xs = sc_gather(x, src_s, E, D)
        ea, v, lg = pl.pallas_call(
            _edge1_body,
            grid=(E // BLK_E,),
            in_specs=[_bspec(BLK_E, D)] * 3 + [
                _cspec(shp_dd), _cspec(shp_dd), _cspec(shp_dd), _cspec((D,)),
                _cspec(shp_dd), _cspec((D,)), _cspec(shp_dd), _cspec(shp_dd),
                _cspec(shp_dd), _cspec((D, 16)),
            ],
            out_specs=[_bspec(BLK_E, D), _bspec(BLK_E, D), _bspec(BLK_E, 16)],
            out_shape=[jax.ShapeDtypeStruct((E, D), f32),
                       jax.ShapeDtypeStruct((E, D), f32),
                       jax.ShapeDtypeStruct((E, 16), f32)],
        )(xd, xs, ea, mW[:D], mW[D:2 * D], mW[2 * D:], p[pre + 'mem_b'],
          p[pre + 'eu_W'], p[pre + 'eu_b'], p[pre + 'q_W'], p[pre + 'k_W'],
          p[pre + 'v_W'], hsel)

        aggr = sc_attn_aggregate(lg, v, dst_pad, e_starts)

        x = _row_call(
            _lane_body, L, BLK_L, [D, D], [D],
            consts=(shp_dd, (D, D_FFN), (D_FFN,), (D_FFN, D), (D,)),
        )(x, aggr, p[pre + 'o_W'], p[pre + 'ffn1_W'], p[pre + 'ffn1_b'],
          p[pre + 'ffn2_W'], p[pre + 'ffn2_b'])

    l2l_attr = sc_scatter(ea, perm, E, D)
    return (x, l2l_attr)


# speedup vs baseline: 2.0021x; 1.0055x over previous
import functools

import jax
import jax.numpy as jnp
import numpy as np
from jax import lax
from jax.experimental import pallas as pl
from jax.experimental.pallas import tpu as pltpu
from jax.experimental.pallas import tpu_sc as plsc

D = 128
H = 8
DH = D // H
L = 10000
NPTS = 100000
E = 320000
IN_DIM = 10
D_RPE = 8
NUM_LAYERS = 2
D_FFN = 2 * D

NC = 2
NS = 16
NW = NC * NS
PLANES = (L + NW - 1) // NW
NEG = np.float32(-3.0e38)

BLK_P = 400
BLK_E = 512
BLK_L = 2000


def _ln(x):
    m = jnp.mean(x, axis=-1, keepdims=True)
    v = jnp.mean((x - m) ** 2, axis=-1, keepdims=True)
    return (x - m) / jnp.sqrt(v + 1e-5)


def _rln(x):
    return jax.nn.relu(_ln(x))


def _cspec(shape):
    return pl.BlockSpec(shape, lambda i, _s=shape: tuple(0 for _ in _s))


def _bspec(blk, width):
    return pl.BlockSpec((blk, width), lambda i: (i, 0))


def _row_call(body, n_rows, blk, in_widths, out_widths, consts):
    grid = (n_rows // blk,)
    in_specs = [_bspec(blk, w) for w in in_widths]
    in_specs += [_cspec(c) for c in consts]
    out_specs = [_bspec(blk, w) for w in out_widths]
    out_shape = [jax.ShapeDtypeStruct((n_rows, w), jnp.float32)
                 for w in out_widths]
    if len(out_widths) == 1:
        out_specs, out_shape = out_specs[0], out_shape[0]
    return pl.pallas_call(body, grid=grid, in_specs=in_specs,
                          out_specs=out_specs, out_shape=out_shape)


_SC_MESH = plsc.VectorSubcoreMesh(core_axis_name="c", subcore_axis_name="s")


def _wid():
    return lax.axis_index("s") * NC + lax.axis_index("c")


def sc_gather(table, idx, n_out, width, chunk=400):
    nch = n_out // (NW * chunk)

    @functools.partial(
        pl.kernel,
        out_type=jax.ShapeDtypeStruct((n_out, width), jnp.float32),
        mesh=_SC_MESH,
        scratch_types=[
            pltpu.VMEM((chunk,), jnp.int32),
            pltpu.VMEM((chunk,), jnp.int32),
            pltpu.VMEM((chunk, width), jnp.float32),
            pltpu.VMEM((chunk, width), jnp.float32),
            pltpu.SemaphoreType.DMA((6,)),
        ],
    )
    def k(table_hbm, idx_hbm, out_hbm, idx_v0, idx_v1, rows_v0, rows_v1,
          sems):
        w = _wid()
        ib = [idx_v0, idx_v1]
        rb = [rows_v0, rows_v1]
        si = [sems.at[0], sems.at[1]]
        sg = [sems.at[2], sems.at[3]]
        so = [sems.at[4], sems.at[5]]

        def base(c):
            return (w * nch + c) * chunk

        def issue_idx(c):
            return pltpu.async_copy(idx_hbm.at[pl.ds(base(c), chunk)],
                                    ib[c % 2], si[c % 2])

        def issue_gather(c):
            return pltpu.async_copy(table_hbm.at[ib[c % 2]], rb[c % 2],
                                    sg[c % 2])

        def issue_out(c):
            return pltpu.async_copy(rb[c % 2],
                                    out_hbm.at[pl.ds(base(c), chunk)],
                                    so[c % 2])

        hi, hg, ho = {}, {}, {}
        hi[0] = issue_idx(0)
        if nch > 1:
            hi[1] = issue_idx(1)
        hi[0].wait()
        hg[0] = issue_gather(0)
        for c in range(nch):
            if c + 1 < nch:
                hi[c + 1].wait()
                if c >= 1:
                    ho[c - 1].wait()
                hg[c + 1] = issue_gather(c + 1)
            hg[c].wait()
            ho[c] = issue_out(c)
            if c + 2 < nch:
                hi[c + 2] = issue_idx(c + 2)
        if nch > 1:
            ho[nch - 2].wait()
        ho[nch - 1].wait()

    return k(table, idx)


def sc_scatter(rows, idx, n_out, width, chunk=400):
    n_in = rows.shape[0]
    nch = n_in // (NW * chunk)

    @functools.partial(
        pl.kernel,
        out_type=jax.ShapeDtypeStruct((n_out, width), jnp.float32),
        mesh=_SC_MESH,
        scratch_types=[
            pltpu.VMEM((chunk,), jnp.int32),
            pltpu.VMEM((chunk,), jnp.int32),
            pltpu.VMEM((chunk, width), jnp.float32),
            pltpu.VMEM((chunk, width), jnp.float32),
            pltpu.SemaphoreType.DMA((6,)),
        ],
    )
    def k(rows_hbm, idx_hbm, out_hbm, idx_v0, idx_v1, rows_v0, rows_v1,
          sems):
        w = _wid()
        ib = [idx_v0, idx_v1]
        rb = [rows_v0, rows_v1]
        si = [sems.at[0], sems.at[1]]
        sr = [sems.at[2], sems.at[3]]
        so = [sems.at[4], sems.at[5]]

        def base(c):
            return (w * nch + c) * chunk

        def issue_in(c):
            return (pltpu.async_copy(idx_hbm.at[pl.ds(base(c), chunk)],
                                     ib[c % 2], si[c % 2]),
                    pltpu.async_copy(rows_hbm.at[pl.ds(base(c), chunk)],
                                     rb[c % 2], sr[c % 2]))

        hi, ho = {}, {}
        hi[0] = issue_in(0)
        for c in range(nch):
            if c + 1 < nch:
                if c >= 1:
                    ho[c - 1].wait()
                hi[c + 1] = issue_in(c + 1)
            hi[c][0].wait()
            hi[c][1].wait()
            ho[c] = pltpu.async_copy(rb[c % 2], out_hbm.at[ib[c % 2]],
                                     so[c % 2])
        if nch > 1:
            ho[nch - 2].wait()
        ho[nch - 1].wait()

    return k(rows, idx)


def sc_seg_reduce(x, seg_pad, starts, n_rows, width, is_max, expand,
                  chunk=256):
    nfb = width // 16
    out_types = [jax.ShapeDtypeStruct((NW * PLANES * width,), jnp.float32)]
    if expand:
        out_types.append(jax.ShapeDtypeStruct((n_rows + 8, width),
                                              jnp.float32))
    ident = NEG if is_max else np.float32(0.0)
    tab_words = (PLANES + 2) * width

    @functools.partial(
        pl.kernel,
        out_type=tuple(out_types) if expand else out_types[0],
        mesh=_SC_MESH,
        scratch_types=[
            pltpu.VMEM((tab_words,), jnp.float32),
            pltpu.VMEM((chunk * width,), jnp.float32),
            pltpu.VMEM((chunk, width), jnp.float32),
            pltpu.VMEM((chunk + 32,), jnp.int32),
            pltpu.VMEM((chunk,), jnp.int32),
            pltpu.VMEM((NW + 16,), jnp.int32),
            pltpu.SemaphoreType.DMA,
        ],
    )
    def k(x_hbm, seg_hbm, starts_hbm, *rest):
        if expand:
            mx_hbm, y_hbm, tab, buf, obuf, segv, yidx, startv, sem = rest
        else:
            mx_hbm, tab, buf, obuf, segv, yidx, startv, sem = rest
        w = _wid()
        pltpu.sync_copy(starts_hbm, startv)
        my_start = startv[pl.ds(w, 16)][0]
        my_end = startv[pl.ds(w + 1, 16)][0]
        lane0 = w * PLANES
        nchunks = lax.div(my_end - my_start + chunk - 1, chunk)

        def init_body(j, _):
            tab[pl.ds(j * 16, 16)] = jnp.full((16,), ident, jnp.float32)
            return 0
        lax.fori_loop(0, tab_words // 16, init_body, 0)

        def load_seg(cc):
            base_u = my_start + cc * chunk
            base = jnp.maximum(jnp.minimum(base_u, my_end - chunk), 0)
            abase = (base // 8) * 8
            pltpu.sync_copy(seg_hbm.at[pl.ds(abase, chunk + 16)],
                            segv.at[pl.ds(0, chunk + 16)])
            return base_u, base, abase

        def _comb(a, b):
            return jnp.maximum(a, b) if is_max else a + b

        def reduce_chunk(cc, _):
            base_u, base, abase = load_seg(cc)
            pltpu.sync_copy(x_hbm.at[pl.ds(base * width, chunk * width)], buf)

            def row(i, carry):
                cur = carry[0]
                acc = carry[1:]
                g = base + i
                sl = segv[pl.ds(i + (base - abase), 16)][0] - lane0
                valid = jnp.logical_and(g >= base_u, g < my_end)
                sl = jnp.where(valid, sl, PLANES)
                changed = sl != cur

                def flush(_):
                    for f in range(nfb):
                        t = tab[pl.ds(cur * width + f * 16, 16)]
                        tab[pl.ds(cur * width + f * 16, 16)] = _comb(t, acc[f])
                    return 0
                lax.cond(changed, flush, lambda _: 0, 0)
                acc3 = tuple(
                    _comb(jnp.where(changed, ident, acc[f]),
                          buf[pl.ds(i * width + f * 16, 16)])
                    for f in range(nfb))
                return (sl,) + acc3

            iacc = tuple(jnp.full((16,), ident, jnp.float32)
                         for _ in range(nfb))
            fin = lax.fori_loop(0, chunk, row, (PLANES + 1,) + iacc,
                                unroll=4)
            cur = fin[0]
            for f in range(nfb):
                t = tab[pl.ds(cur * width + f * 16, 16)]
                tab[pl.ds(cur * width + f * 16, 16)] = _comb(t, fin[1 + f])
            return 0
        lax.fori_loop(0, nchunks, reduce_chunk, 0)

        if is_max:
            def fix_body(j, _):
                t = tab[pl.ds(j * 16, 16)]
                tab[pl.ds(j * 16, 16)] = jnp.where(t <= NEG, 0.0, t)
                return 0
            lax.fori_loop(0, tab_words // 16, fix_body, 0)

        pltpu.sync_copy(tab.at[pl.ds(0, PLANES * width)],
                        mx_hbm.at[pl.ds(lane0 * width, PLANES * width)])

        if expand:
            def expand_chunk(cc, _):
                base_u, base, abase = load_seg(cc)
                for b in range(chunk // 16):
                    giota = base + b * 16 + lax.iota(jnp.int32, 16)
                    okv = jnp.logical_and(giota >= base_u, giota < my_end)
                    yidx[pl.ds(b * 16, 16)] = jnp.where(okv, giota, n_rows)

                def row(i, _c):
                    sl = jnp.clip(segv[pl.ds(i + (base - abase), 16)][0]
                                  - lane0, 0, PLANES)
                    for f in range(nfb):
                        obuf[i, pl.ds(f * 16, 16)] = tab[
                            pl.ds(sl * width + f * 16, 16)]
                    return _c

                lax.fori_loop(0, chunk, row, 0, unroll=4)
                pltpu.async_copy(obuf, y_hbm.at[yidx], sem).wait()
                return 0
            lax.fori_loop(0, nchunks, expand_chunk, 0)

    outs = k(x.reshape(-1), seg_pad, starts)
    if expand:
        mx, y = outs
        return mx.reshape(NW * PLANES, width)[:L], y[:n_rows]
    return outs.reshape(NW * PLANES, width)[:L]


def sc_attn_aggregate(lg, v, seg_pad, starts, chunk=384):
    tabw = (PLANES + 2) * 16
    taga = (PLANES + 2) * D

    @functools.partial(
        pl.kernel,
        out_type=jax.ShapeDtypeStruct((NW * PLANES * D,), jnp.float32),
        mesh=_SC_MESH,
        scratch_types=[
            pltpu.VMEM((tabw,), jnp.float32),
            pltpu.VMEM((tabw,), jnp.float32),
            pltpu.VMEM((taga,), jnp.float32),
            pltpu.VMEM((chunk * 16,), jnp.float32),
            pltpu.VMEM((chunk * D,), jnp.float32),
            pltpu.VMEM((chunk + 32,), jnp.int32),
            pltpu.VMEM((NW + 16,), jnp.int32),
        ],
    )
    def k(lg_hbm, v_hbm, seg_hbm, starts_hbm, out_hbm,
          mtab, stab, atab, lbuf, vbuf, segv, startv):
        w = _wid()
        pltpu.sync_copy(starts_hbm, startv)
        my_start = startv[pl.ds(w, 16)][0]
        my_end = startv[pl.ds(w + 1, 16)][0]
        lane0 = w * PLANES
        nchunks = lax.div(my_end - my_start + chunk - 1, chunk)

        def init16(j, _):
            mtab[pl.ds(j * 16, 16)] = jnp.full((16,), NEG, jnp.float32)
            stab[pl.ds(j * 16, 16)] = jnp.zeros((16,), jnp.float32)
            return 0
        lax.fori_loop(0, tabw // 16, init16, 0)

        def inita(j, _):
            atab[pl.ds(j * 16, 16)] = jnp.zeros((16,), jnp.float32)
            return 0
        lax.fori_loop(0, taga // 16, inita, 0)

        def load_seg(cc):
            base_u = my_start + cc * chunk
            base = jnp.maximum(jnp.minimum(base_u, my_end - chunk), 0)
            abase = (base // 8) * 8
            pltpu.sync_copy(seg_hbm.at[pl.ds(abase, chunk + 16)],
                            segv.at[pl.ds(0, chunk + 16)])
            return base_u, base, abase

        def rowmeta(base_u, base, abase, i):
            g = base + i
            sl = segv[pl.ds(i + (base - abase), 16)][0] - lane0
            valid = jnp.logical_and(g >= base_u, g < my_end)
            return jnp.where(valid, sl, PLANES)

        def pass_ms(cc, _):
            base_u, base, abase = load_seg(cc)
            pltpu.sync_copy(lg_hbm.at[pl.ds(base * 16, chunk * 16)], lbuf)

            def merge(cur, macc, sacc):
                mo = mtab[pl.ds(cur * 16, 16)]
                so = stab[pl.ds(cur * 16, 16)]
                mm = jnp.maximum(mo, macc)
                stab[pl.ds(cur * 16, 16)] = (so * jnp.exp(mo - mm)
                                             + sacc * jnp.exp(macc - mm))
                mtab[pl.ds(cur * 16, 16)] = mm

            def row(i, carry):
                cur, macc, sacc = carry
                sl = rowmeta(base_u, base, abase, i)
                changed = sl != cur

                def flush(_):
                    merge(cur, macc, sacc)
                    return 0
                lax.cond(changed, flush, lambda _: 0, 0)
                m2 = jnp.where(changed, NEG, macc)
                s2 = jnp.where(changed, 0.0, sacc)
                lv = lbuf[pl.ds(i * 16, 16)]
                mn = jnp.maximum(m2, lv)
                s3 = s2 * jnp.exp(m2 - mn) + jnp.exp(lv - mn)
                return (sl, mn, s3)

            cur, macc, sacc = lax.fori_loop(
                0, chunk, row,
                (PLANES + 1, jnp.full((16,), NEG, jnp.float32),
                 jnp.zeros((16,), jnp.float32)), unroll=4)
            merge(cur, macc, sacc)
            return 0
        lax.fori_loop(0, nchunks, pass_ms, 0)

        def recip(j, _):
            t = stab[pl.ds(j * 16, 16)]
            stab[pl.ds(j * 16, 16)] = 1.0 / (t + 1e-16)
            return 0
        lax.fori_loop(0, tabw // 16, recip, 0)

        def pass_wv(cc, _):
            base_u, base, abase = load_seg(cc)
            pltpu.sync_copy(lg_hbm.at[pl.ds(base * 16, chunk * 16)], lbuf)
            pltpu.sync_copy(v_hbm.at[pl.ds(base * D, chunk * D)], vbuf)

            def row(i, carry):
                cur = carry[0]
                acc = carry[1:]
                sl = rowmeta(base_u, base, abase, i)
                changed = sl != cur

                def flush(_):
                    for h in range(H):
                        a = atab[pl.ds(cur * D + h * 16, 16)]
                        atab[pl.ds(cur * D + h * 16, 16)] = a + acc[h]
                    return 0
                lax.cond(changed, flush, lambda _: 0, 0)
                mv = mtab[pl.ds(sl * 16, 16)]
                sv = stab[pl.ds(sl * 16, 16)]
                wt = jnp.exp(lbuf[pl.ds(i * 16, 16)] - mv) * sv
                acc3 = tuple(
                    jnp.where(changed, 0.0, acc[h])
                    + vbuf[pl.ds(i * D + h * 16, 16)] * wt[h]
                    for h in range(H))
                return (sl,) + acc3

            zero = jnp.zeros((16,), jnp.float32)
            fin = lax.fori_loop(
                0, chunk, row,
                (PLANES + 1,) + tuple(zero for _ in range(H)), unroll=4)
            cur = fin[0]
            for h in range(H):
                a = atab[pl.ds(cur * D + h * 16, 16)]
                atab[pl.ds(cur * D + h * 16, 16)] = a + fin[1 + h]
            return 0
        lax.fori_loop(0, nchunks, pass_wv, 0)

        pltpu.sync_copy(atab.at[pl.ds(0, PLANES * D)],
                        out_hbm.at[pl.ds(lane0 * D, PLANES * D)])

    out = k(lg.reshape(-1), v.reshape(-1), seg_pad, starts)
    return out.reshape(NW * PLANES, D)[:L]


def _points1_body(nf, projW, projB, f1aW, f1aB, f1bW, f1bB, x0o, x1o):
    x0 = _rln(jnp.dot(nf[...], projW[...]) + projB[...])
    t = _rln(jnp.dot(x0, f1aW[...]) + f1aB[...])
    x1 = _rln(jnp.dot(t, f1bW[...]) + f1bB[...])
    x0o[...] = x0
    x1o[...] = x1


def _fc2_body(xin, x1, y1, aWt, aWb, aB, bW, bB, f1aW, f1aB, f1bW, f1bB,
              outo, x2o):
    h = _rln(jnp.dot(x1[...], aWt[...]) + jnp.dot(y1[...], aWb[...]) + aB[...])
    h = _rln(jnp.dot(h, bW[...]) + bB[...])
    out = _ln(xin[...] + h)
    t = _rln(jnp.dot(out, f1aW[...]) + f1aB[...])
    x2 = _rln(jnp.dot(t, f1bW[...]) + f1bB[...])
    outo[...] = out
    x2o[...] = x2


def _fc2_final_body(xin, x1, y1, aWt, aWb, aB, bW, bB, outo):
    h = _rln(jnp.dot(x1[...], aWt[...]) + jnp.dot(y1[...], aWb[...]) + aB[...])
    h = _rln(jnp.dot(h, bW[...]) + bB[...])
    outo[...] = _ln(xin[...] + h)


def _rpe_body(rp, rW, rB, eao):
    eao[...] = _rln(jnp.dot(rp[...], rW[...]) + rB[...])


def _edge1_body(xd, xs, ea, W1, W2, W3, mB, euW, euB, qW, kW, vW, hsel,
                eao, vo, lo):
    mem = _rln(jnp.dot(xd[...], W1[...]) + jnp.dot(xs[...], W2[...])
               + jnp.dot(ea[...], W3[...]) + mB[...])
    delta = _rln(jnp.dot(mem, euW[...]) + euB[...])
    eao[...] = _ln(ea[...] + delta)
    q = jnp.dot(xd[...], qW[...])
    kk = jnp.dot(mem, kW[...])
    vo[...] = jnp.dot(mem, vW[...])
    lo[...] = jnp.dot(q * kk, hsel[...]) * (1.0 / np.sqrt(DH))


def _lane_body(x, aggr, oW, f1W, f1B, f2W, f2B, xo):
    x1 = _ln(x[...] + jnp.dot(aggr[...], oW[...]))
    h = jax.nn.relu(jnp.dot(x1, f1W[...]) + f1B[...])
    xo[...] = _ln(x1 + jnp.dot(h, f2W[...]) + f2B[...])


def kernel(node_feats, nodes_of_lanes, l2l_edges, l2l_fused_rpes, params):
    p = params
    f32 = jnp.float32

    src, dst = l2l_edges[0], l2l_edges[1]
    perm = jnp.argsort(dst).astype(jnp.int32)
    dst_s = dst[perm]
    src_s = src[perm]
    lane_cuts = jnp.arange(NW + 16, dtype=jnp.int32) * PLANES
    e_starts = jnp.minimum(
        jnp.searchsorted(dst_s, lane_cuts), E).astype(jnp.int32)
    n_starts = jnp.minimum(
        jnp.searchsorted(nodes_of_lanes, lane_cuts), NPTS).astype(jnp.int32)
    nol_pad = jnp.concatenate([nodes_of_lanes, jnp.full((16,), L, jnp.int32)])
    dst_pad = jnp.concatenate([dst_s, jnp.full((16,), L, jnp.int32)])

    nf_pad = jnp.zeros((NPTS, D), f32).at[:, :IN_DIM].set(node_feats)
    projW = jnp.zeros((D, D), f32).at[:IN_DIM].set(p['proj_W'])

    shp_dd = (D, D)
    x0, x1 = _row_call(
        _points1_body, NPTS, BLK_P, [D], [D, D],
        consts=(shp_dd, (D,), shp_dd, (D,), shp_dd, (D,)),
    )(nf_pad, projW, p['proj_b'], p['pa_fc1a_W'], p['pa_fc1a_b'],
      p['pa_fc1b_W'], p['pa_fc1b_b'])

    _, y1 = sc_seg_reduce(x1, nol_pad, n_starts, NPTS, D,
                          is_max=True, expand=True)

    out_pa, x2 = _row_call(
        _fc2_body, NPTS, BLK_P, [D, D, D], [D, D],
        consts=(shp_dd, shp_dd, (D,), shp_dd, (D,), shp_dd, (D,),
                shp_dd, (D,)),
    )(x0, x1, y1, p['pa_fc2a_W'][:D], p['pa_fc2a_W'][D:], p['pa_fc2a_b'],
      p['pa_fc2b_W'], p['pa_fc2b_b'],
      p['la_fc1a_W'], p['la_fc1a_b'], p['la_fc1b_W'], p['la_fc1b_b'])

    _, y2 = sc_seg_reduce(x2, nol_pad, n_starts, NPTS, D,
                          is_max=True, expand=True)

    out_la = _row_call(
        _fc2_final_body, NPTS, BLK_P, [D, D, D], [D],
        consts=(shp_dd, shp_dd, (D,), shp_dd, (D,)),
    )(out_pa, x2, y2, p['la_fc2a_W'][:D], p['la_fc2a_W'][D:],
      p['la_fc2a_b'], p['la_fc2b_W'], p['la_fc2b_b'])

    x = sc_seg_reduce(out_la, nol_pad, n_starts, NPTS, D,
                      is_max=True, expand=False)

    rpW = jnp.zeros((16, D), f32).at[:D_RPE].set(p['rpe_W'])
    rpes_p = jnp.zeros((E, 16), f32).at[:, :D_RPE].set(l2l_fused_rpes)
    ea0 = _row_call(
        _rpe_body, E, BLK_E, [16], [D],
        consts=((16, D), (D,)),
    )(rpes_p, rpW, p['rpe_b'])
    ea = sc_gather(ea0, perm, E, D)

    hsel_np = np.zeros((D, 16), np.float32)
    for h in range(H):
        hsel_np[h * DH:(h + 1) * DH, h] = 1.0
    hsel = jnp.asarray(hsel_np)

    for l in range(NUM_LAYERS):
        pre = 'l%d_' % l
        mW = p[pre + 'mem_W']
        xd = sc_gather(x, dst_s, E, D)
        xs = sc_gather(x, src_s, E, D)
        ea, v, lg = pl.pallas_call(
            _edge1_body,
            grid=(E // BLK_E,),
            in_specs=[_bspec(BLK_E, D)] * 3 + [
                _cspec(shp_dd), _cspec(shp_dd), _cspec(shp_dd), _cspec((D,)),
                _cspec(shp_dd), _cspec((D,)), _cspec(shp_dd), _cspec(shp_dd),
                _cspec(shp_dd), _cspec((D, 16)),
            ],
            out_specs=[_bspec(BLK_E, D), _bspec(BLK_E, D), _bspec(BLK_E, 16)],
            out_shape=[jax.ShapeDtypeStruct((E, D), f32),
                       jax.ShapeDtypeStruct((E, D), f32),
                       jax.ShapeDtypeStruct((E, 16), f32)],
        )(xd, xs, ea, mW[:D], mW[D:2 * D], mW[2 * D:], p[pre + 'mem_b'],
          p[pre + 'eu_W'], p[pre + 'eu_b'], p[pre + 'q_W'], p[pre + 'k_W'],
          p[pre + 'v_W'], hsel)

        aggr = sc_attn_aggregate(lg, v, dst_pad, e_starts)

        x = _row_call(
            _lane_body, L, BLK_L, [D, D], [D],
            consts=(shp_dd, (D, D_FFN), (D_FFN,), (D_FFN, D), (D,)),
        )(x, aggr, p[pre + 'o_W'], p[pre + 'ffn1_W'], p[pre + 'ffn1_b'],
          p[pre + 'ffn2_W'], p[pre + 'ffn2_b'])

    l2l_attr = sc_scatter(ea, perm, E, D)
    return (x, l2l_attr)
